# Initial kernel scaffold; baseline (speedup 1.0000x reference)
import jax
import jax.numpy as jnp
from jax.experimental import pallas as pl


def kernel(x, params, edge_index):
    raise NotImplementedError("write your pallas kernel here")


# speedup vs baseline: 14.3107x; 14.3107x over previous
import functools

import jax
import jax.numpy as jnp
from jax import lax
from jax.experimental import pallas as pl
from jax.experimental.pallas import tpu as pltpu
from jax.experimental.pallas import tpu_sc as plsc

N = 10000
D = 128
MID = 512
E = 320000
ETOT = E + N
NTILE = 32
CH = 128
T_PER_TILE = 10496
NCHUNK = T_PER_TILE // CH
TOT = NTILE * T_PER_TILE
EROWS = TOT // 128
TROWS = T_PER_TILE // 128
NPAD = 10112
ZROWS = NPAD // 16

R = 400
GRID = N // R

_f32 = jnp.float32
_i32 = jnp.int32


def _sc_conv_body(src_hbm, dst_hbm, asrc_hbm, adst_hbm, c_hbm, h_hbm,
                  acc_out, den_out,
                  asrc_v, adst_v, den_v, src2d, dst2d, ee_v, rows_v, c_v,
                  acc_s):
    cid = lax.axis_index("c")
    sid = lax.axis_index("s")
    wid = cid * 16 + sid

    pltpu.sync_copy(asrc_hbm, asrc_v)
    pltpu.sync_copy(adst_hbm, adst_v)
    pltpu.sync_copy(c_hbm, c_v)
    cvec = c_v[...]

    zf = jnp.zeros((16,), _f32)

    def _zr(r, carry):
        for f in range(8):
            rows_v[r, pl.ds(f * 16, 16)] = zf
        return carry
    lax.fori_loop(0, CH, _zr, 0)

    def _zd(i, carry):
        den_v[pl.ds(pl.multiple_of(i * 16, 16), 16)] = zf
        return carry
    lax.fori_loop(0, NPAD // 16, _zd, 0)

    zbase = pl.multiple_of(sid * ZROWS, 8)
    nfull = ZROWS // CH
    for k in range(nfull):
        pltpu.sync_copy(rows_v, acc_s.at[pl.ds(zbase + k * CH, CH)])
    rem = ZROWS - nfull * CH
    if rem:
        pltpu.sync_copy(rows_v.at[pl.ds(0, rem)],
                        acc_s.at[pl.ds(zbase + nfull * CH, rem)])
    plsc.subcore_barrier()

    def _chunk(ch, carry):
        brow = wid * TROWS + ch * (CH // 128)
        pltpu.sync_copy(src_hbm.at[pl.ds(brow, CH // 128)], src2d)
        pltpu.sync_copy(dst_hbm.at[pl.ds(brow, CH // 128)], dst2d)
        for r in range(CH // 128):
            def _sp(cc, carry2, _r=r):
                off = pl.multiple_of(cc * 16, 16)
                s16 = src2d[_r, pl.ds(off, 16)]
                d16 = dst2d[_r, pl.ds(off, 16)]
                av = plsc.load_gather(asrc_v, [s16])
                bv = plsc.load_gather(adst_v, [d16])
                e = av + bv
                e = jnp.maximum(e, 0.2 * e)
                ee = jnp.exp(e - cvec)
                ee_v[pl.ds(pl.multiple_of(_r * 128 + cc * 16, 16), 16)] = ee
                plsc.addupdate_scatter(den_v, [d16], ee)
                return carry2
            lax.fori_loop(0, 8, _sp, 0)
            pltpu.sync_copy(h_hbm.at[src2d.at[r]],
                            rows_v.at[pl.ds(r * 128, 128)])
        def _scale(ei, carry2):
            eev = plsc.load_gather(ee_v, [jnp.zeros((16,), _i32) + ei])
            for f in range(8):
                sl = pl.ds(f * 16, 16)
                rows_v[ei, sl] = rows_v[ei, sl] * eev
            return carry2
        lax.fori_loop(0, CH, _scale, 0)
        for r in range(CH // 128):
            pltpu.sync_copy(rows_v.at[pl.ds(r * 128, 128)],
                            acc_s.at[dst2d.at[r]], add=True)
        return carry
    lax.fori_loop(0, NCHUNK, _chunk, 0)

    plsc.subcore_barrier()
    obase = pl.multiple_of(sid * ZROWS, 8)
    pltpu.sync_copy(acc_s.at[pl.ds(obase, ZROWS)],
                    acc_out.at[cid, pl.ds(obase, ZROWS)])
    pltpu.sync_copy(den_v, den_out.at[pl.ds(pl.multiple_of(wid * NPAD, 128),
                                            NPAD)])


_sc_conv = functools.partial(
    pl.kernel,
    out_type=(jax.ShapeDtypeStruct((2, NPAD, D), _f32),
              jax.ShapeDtypeStruct((NTILE * NPAD,), _f32)),
    mesh=plsc.VectorSubcoreMesh(core_axis_name="c", subcore_axis_name="s",
                                num_cores=2, num_subcores=16),
    compiler_params=pltpu.CompilerParams(needs_layout_passes=False),
    scratch_types=(
        pltpu.VMEM((NPAD,), _f32),
        pltpu.VMEM((NPAD,), _f32),
        pltpu.VMEM((NPAD,), _f32),
        pltpu.VMEM((CH // 128, 128), _i32),
        pltpu.VMEM((CH // 128, 128), _i32),
        pltpu.VMEM((CH,), _f32),
        pltpu.VMEM((CH, D), _f32),
        pltpu.VMEM((16,), _f32),
        pltpu.VMEM_SHARED((NPAD, D), _f32),
    ),
)(_sc_conv_body)


def _erf(z):
    a = jnp.abs(z)
    t = 1.0 / (1.0 + 0.3275911 * a)
    poly = t * (0.254829592 + t * (-0.284496736 + t * (1.421413741
                + t * (-1.453152027 + t * 1.061405429))))
    return jnp.sign(z) * (1.0 - poly * jnp.exp(-a * a))


def _gelu(v):
    return 0.5 * v * (1.0 + _erf(v * 0.7071067811865476))


def _ln(v, g, b):
    mu = jnp.mean(v, axis=-1, keepdims=True)
    c = v - mu
    var = jnp.mean(c * c, axis=-1, keepdims=True)
    return c * lax.rsqrt(var + 1e-12) * g + b


def _dot(a, b):
    return jnp.dot(a, b, preferred_element_type=_f32)


def _conv_prep(x2, gw_ref, gas_ref, gad_ref, h_ref, as_ref, ad_ref,
               mas_ref, mad_ref, first):
    h = _dot(x2, gw_ref[...])
    h_ref[...] = h
    a_s = jnp.sum(h * gas_ref[...], axis=-1, keepdims=True)
    a_d = jnp.sum(h * gad_ref[...], axis=-1, keepdims=True)
    as_ref[...] = a_s
    ad_ref[...] = a_d

    @pl.when(first)
    def _():
        mas_ref[...] = jnp.full((1, 1), -1e30, _f32)
        mad_ref[...] = jnp.full((1, 1), -1e30, _f32)
    mas_ref[...] = jnp.maximum(mas_ref[...], jnp.max(a_s))
    mad_ref[...] = jnp.maximum(mad_ref[...], jnp.max(a_d))


def _stage_a_body(x_ref, wpre_ref, bpre_ref,
                  gw0_ref, gas0_ref, gad0_ref, gw1_ref, gas1_ref, gad1_ref,
                  x1_ref, h0_ref, as0_ref, ad0_ref, h1_ref, as1_ref, ad1_ref,
                  mas0_ref, mad0_ref, mas1_ref, mad1_ref):
    x = x_ref[...]
    t = _dot(x, wpre_ref[...]) + bpre_ref[...]
    x1 = jnp.maximum(t, 0.01 * t)
    x1_ref[...] = x1
    first = pl.program_id(0) == 0
    _conv_prep(x1, gw0_ref, gas0_ref, gad0_ref, h0_ref, as0_ref, ad0_ref,
               mas0_ref, mad0_ref, first)
    _conv_prep(x1, gw1_ref, gas1_ref, gad1_ref, h1_ref, as1_ref, ad1_ref,
               mas1_ref, mad1_ref, first)


def _stage_b_body(final, prex_ref, acc0_ref, den0_ref, acc1_ref, den1_ref,
                  gb0_ref, gb1_ref, wq_ref, bq_ref, wk_ref, bk_ref,
                  wv_ref, bv_ref, lng_ref, lnb_ref,
                  wm_ref, bm_ref, wo_ref, bo_ref, olg_ref, olb_ref,
                  *tail):
    prex = prex_ref[...]

    def _xt(acc_ref, den_ref, gb_ref):
        a = acc_ref[...]
        dsum = jnp.sum(den_ref[...], axis=-1, keepdims=True)
        return (a[0] + a[1]) / dsum + gb_ref[...]

    xt0 = _xt(acc0_ref, den0_ref, gb0_ref)
    xt1 = _xt(acc1_ref, den1_ref, gb1_ref)

    q = _dot(prex, wq_ref[...]) + bq_ref[...]
    k0 = _dot(xt0, wk_ref[...]) + bk_ref[...]
    k1 = _dot(xt1, wk_ref[...]) + bk_ref[...]
    l0 = jnp.sum(q * k0, axis=-1, keepdims=True)
    l1 = jnp.sum(q * k1, axis=-1, keepdims=True)
    m = jnp.maximum(l0, l1)
    w0 = jnp.exp(l0 - m)
    w1 = jnp.exp(l1 - m)
    v0 = _dot(xt0, wv_ref[...]) + bv_ref[...]
    v1 = _dot(xt1, wv_ref[...]) + bv_ref[...]
    xatt = (w0 * v0 + w1 * v1) / (w0 + w1)

    x = _ln(prex + _gelu(xatt), lng_ref[...], lnb_ref[...])
    midv = _gelu(_dot(x, wm_ref[...]) + bm_ref[...])
    mid2 = _dot(midv, wo_ref[...]) + bo_ref[...]
    x2 = _ln(x + mid2, olg_ref[...], olb_ref[...])

    if final:
        wfin_ref, bfin_ref, out_ref = tail
        out_ref[...] = _gelu(_dot(x2, wfin_ref[...]) + bfin_ref[...])
    else:
        (gw0_ref, gas0_ref, gad0_ref, gw1_ref, gas1_ref, gad1_ref,
         x2_ref, h0_ref, as0_ref, ad0_ref, h1_ref, as1_ref, ad1_ref,
         mas0_ref, mad0_ref, mas1_ref, mad1_ref) = tail
        x2_ref[...] = x2
        first = pl.program_id(0) == 0
        _conv_prep(x2, gw0_ref, gas0_ref, gad0_ref, h0_ref, as0_ref, ad0_ref,
                   mas0_ref, mad0_ref, first)
        _conv_prep(x2, gw1_ref, gas1_ref, gad1_ref, h1_ref, as1_ref, ad1_ref,
                   mas1_ref, mad1_ref, first)


def _blk(shape, idx):
    return pl.BlockSpec(shape, idx)


_ROWB = _blk((R, D), lambda i: (i, 0))
_W128 = _blk((D, D), lambda i: (0, 0))
_ROW1 = _blk((1, D), lambda i: (0, 0))
_COL1 = _blk((R, 1), lambda i: (i, 0))
_SCLR = _blk((1, 1), lambda i: (0, 0))

_CONV_OUT_SHAPES = (
    jax.ShapeDtypeStruct((N, D), _f32),
    jax.ShapeDtypeStruct((N, 1), _f32),
    jax.ShapeDtypeStruct((N, 1), _f32),
)
_CONV_OUT_SPECS = (_ROWB, _COL1, _COL1)
_MAX_OUT = (jax.ShapeDtypeStruct((1, 1), _f32),) * 2
_MAX_SPEC = (_SCLR, _SCLR)


def _stage_a(x, wpre, bpre, gw0, gas0, gad0, gw1, gas1, gad1):
    return pl.pallas_call(
        _stage_a_body,
        grid=(GRID,),
        in_specs=[_ROWB, _W128, _ROW1,
                  _W128, _ROW1, _ROW1, _W128, _ROW1, _ROW1],
        out_specs=(_ROWB,) + _CONV_OUT_SPECS + _CONV_OUT_SPECS
        + _MAX_SPEC + _MAX_SPEC,
        out_shape=(jax.ShapeDtypeStruct((N, D), _f32),)
        + _CONV_OUT_SHAPES + _CONV_OUT_SHAPES + _MAX_OUT + _MAX_OUT,
    )(x, wpre, bpre, gw0, gas0, gad0, gw1, gas1, gad1)


_ACCB = _blk((2, R, D), lambda i: (0, i, 0))
_DENB = _blk((R, NTILE), lambda i: (i, 0))
_WMID = _blk((D, MID), lambda i: (0, 0))
_ROWM = _blk((1, MID), lambda i: (0, 0))
_WOUT = _blk((MID, D), lambda i: (0, 0))

_B_COMMON_SPECS = [
    _ROWB, _ACCB, _DENB, _ACCB, _DENB,
    _ROW1, _ROW1, _W128, _ROW1, _W128, _ROW1, _W128, _ROW1,
    _ROW1, _ROW1, _WMID, _ROWM, _WOUT, _ROW1, _ROW1, _ROW1,
]


def _stage_b_mid(args, tail_weights):
    return pl.pallas_call(
        functools.partial(_stage_b_body, False),
        grid=(GRID,),
        in_specs=_B_COMMON_SPECS + [_W128, _ROW1, _ROW1, _W128, _ROW1, _ROW1],
        out_specs=(_ROWB,) + _CONV_OUT_SPECS + _CONV_OUT_SPECS
        + _MAX_SPEC + _MAX_SPEC,
        out_shape=(jax.ShapeDtypeStruct((N, D), _f32),)
        + _CONV_OUT_SHAPES + _CONV_OUT_SHAPES + _MAX_OUT + _MAX_OUT,
    )(*args, *tail_weights)


def _stage_b_fin(args, tail_weights):
    return pl.pallas_call(
        functools.partial(_stage_b_body, True),
        grid=(GRID,),
        in_specs=_B_COMMON_SPECS + [_W128, _ROW1],
        out_specs=_ROWB,
        out_shape=jax.ShapeDtypeStruct((N, D), _f32),
    )(*args, *tail_weights)


def _edge_arrays(edge_index):
    ar = jnp.arange(N, dtype=_i32)
    pad_s = jnp.zeros((TOT - ETOT,), _i32)
    pad_d = jnp.full((TOT - ETOT,), N, _i32)
    out = []
    for j in range(2):
        src = jnp.concatenate([edge_index[j, 0].astype(_i32), ar, pad_s])
        dst = jnp.concatenate([edge_index[j, 1].astype(_i32), ar, pad_d])
        out.append((src.reshape(EROWS, 128), dst.reshape(EROWS, 128)))
    return out


def _tables(a_s, a_d, mas, mad):
    c0 = mas[0, 0] + mad[0, 0]
    c = jnp.maximum(c0, 0.2 * c0)
    c16 = jnp.broadcast_to(c, (16,))
    asp = jnp.pad(a_s[:, 0], (0, NPAD - N))
    adp = jnp.pad(a_d[:, 0], (0, NPAD - N))
    return asp, adp, c16


def _run_conv(edges, a_s, a_d, mas, mad, h):
    asp, adp, c16 = _tables(a_s, a_d, mas, mad)
    acc, den = _sc_conv(edges[0], edges[1], asp, adp, c16, h)
    return acc[:, :N, :], den.reshape(NTILE, NPAD)[:, :N].T


def kernel(x, params, edge_index):
    p = params

    def rowv(v):
        return v.reshape(1, -1).astype(_f32)

    edges = _edge_arrays(edge_index)

    def conv_w(i, j):
        return (p['gW_%d_%d' % (i, j)], rowv(p['gas_%d_%d' % (i, j)]),
                rowv(p['gad_%d_%d' % (i, j)]))

    (x1, h00, as00, ad00, h01, as01, ad01,
     mas00, mad00, mas01, mad01) = _stage_a(
        x, p['Wpre'], rowv(p['bpre']), *conv_w(0, 0), *conv_w(0, 1))

    acc00, den00 = _run_conv(edges[0], as00, ad00, mas00, mad00, h00)
    acc01, den01 = _run_conv(edges[1], as01, ad01, mas01, mad01, h01)

    def layer_args(i, prex, acc0, den0, acc1, den1):
        temp = p['atemp_%d' % i]
        return (prex, acc0, den0, acc1, den1,
                rowv(p['gb_%d_0' % i]), rowv(p['gb_%d_1' % i]),
                p['aWq_%d' % i] * temp, rowv(p['abq_%d' % i]) * temp,
                p['aWk_%d' % i], rowv(p['abk_%d' % i]),
                p['aWv_%d' % i], rowv(p['abv_%d' % i]),
                rowv(p['ln_g']), rowv(p['ln_b']),
                p['oWm_%d' % i], rowv(p['obm_%d' % i]),
                p['oWo_%d' % i], rowv(p['obo_%d' % i]),
                rowv(p['olg_%d' % i]), rowv(p['olb_%d' % i]))

    (x2, h10, as10, ad10, h11, as11, ad11,
     mas10, mad10, mas11, mad11) = _stage_b_mid(
        layer_args(0, x1, acc00, den00, acc01, den01),
        conv_w(1, 0) + conv_w(1, 1))

    acc10, den10 = _run_conv(edges[0], as10, ad10, mas10, mad10, h10)
    acc11, den11 = _run_conv(edges[1], as11, ad11, mas11, mad11, h11)

    return _stage_b_fin(
        layer_args(1, x2, acc10, den10, acc11, den11),
        (p['Wfin'], rowv(p['bfin'])))


# speedup vs baseline: 22.2247x; 1.5530x over previous
import functools

import jax
import jax.numpy as jnp
from jax import lax
from jax.experimental import pallas as pl
from jax.experimental.pallas import tpu as pltpu
from jax.experimental.pallas import tpu_sc as plsc

N = 10000
D = 128
MID = 512
E = 320000
ETOT = E + N
NTILE = 32
CH = 128
T_PER_TILE = 20736
NCHUNK = T_PER_TILE // CH
TOT = 16 * T_PER_TILE
EROWS = TOT // 128
TROWS = T_PER_TILE // 128
NPAD = 10112
ZROWS = NPAD // 16

R = 400
GRID = N // R

_f32 = jnp.float32
_i32 = jnp.int32


def _sc_conv_body(src_hbm, dst_hbm, asrc_hbm, adst_hbm, c_hbm, h_hbm,
                  acc_out, den_out,
                  asrc_v, adst_v, den_v, src2d, dst2d, ee_v, rows_v, c_v,
                  acc_s):
    cid = lax.axis_index("c")
    sid = lax.axis_index("s")
    wid = cid * 16 + sid

    pltpu.sync_copy(asrc_hbm.at[cid], asrc_v)
    pltpu.sync_copy(adst_hbm.at[cid], adst_v)
    pltpu.sync_copy(c_hbm.at[cid], c_v)
    cvec = c_v[...]

    zf = jnp.zeros((16,), _f32)

    def _zr(r, carry):
        for f in range(8):
            rows_v[r, pl.ds(f * 16, 16)] = zf
        return carry
    lax.fori_loop(0, CH, _zr, 0)

    def _zd(i, carry):
        den_v[pl.ds(pl.multiple_of(i * 16, 16), 16)] = zf
        return carry
    lax.fori_loop(0, NPAD // 16, _zd, 0)

    zbase = pl.multiple_of(sid * ZROWS, 8)
    nfull = ZROWS // CH
    for k in range(nfull):
        pltpu.sync_copy(rows_v, acc_s.at[pl.ds(zbase + k * CH, CH)])
    rem = ZROWS - nfull * CH
    if rem:
        pltpu.sync_copy(rows_v.at[pl.ds(0, rem)],
                        acc_s.at[pl.ds(zbase + nfull * CH, rem)])
    plsc.subcore_barrier()

    def _chunk(ch, carry):
        brow = sid * TROWS + ch * (CH // 128)
        pltpu.sync_copy(src_hbm.at[cid, pl.ds(brow, CH // 128)], src2d)
        pltpu.sync_copy(dst_hbm.at[cid, pl.ds(brow, CH // 128)], dst2d)
        for r in range(CH // 128):
            def _sp(cc, carry2, _r=r):
                off = pl.multiple_of(cc * 16, 16)
                s16 = src2d[_r, pl.ds(off, 16)]
                d16 = dst2d[_r, pl.ds(off, 16)]
                av = plsc.load_gather(asrc_v, [s16])
                bv = plsc.load_gather(adst_v, [d16])
                e = av + bv
                e = jnp.maximum(e, 0.2 * e)
                ee = jnp.exp(e - cvec)
                ee_v[pl.ds(pl.multiple_of(_r * 128 + cc * 16, 16), 16)] = ee
                plsc.addupdate_scatter(den_v, [d16], ee)
                return carry2
            lax.fori_loop(0, 8, _sp, 0)
            pltpu.sync_copy(h_hbm.at[cid].at[src2d.at[r]],
                            rows_v.at[pl.ds(r * 128, 128)])
        def _scale(ei, carry2):
            eev = plsc.load_gather(ee_v, [jnp.zeros((16,), _i32) + ei])
            for f in range(8):
                sl = pl.ds(f * 16, 16)
                rows_v[ei, sl] = rows_v[ei, sl] * eev
            return carry2
        lax.fori_loop(0, CH, _scale, 0)
        for r in range(CH // 128):
            pltpu.sync_copy(rows_v.at[pl.ds(r * 128, 128)],
                            acc_s.at[dst2d.at[r]], add=True)
        return carry
    lax.fori_loop(0, NCHUNK, _chunk, 0)

    plsc.subcore_barrier()
    obase = pl.multiple_of(sid * ZROWS, 8)
    pltpu.sync_copy(acc_s.at[pl.ds(obase, ZROWS)],
                    acc_out.at[cid, pl.ds(obase, ZROWS)])
    pltpu.sync_copy(den_v, den_out.at[pl.ds(pl.multiple_of(wid * NPAD, 128),
                                            NPAD)])


_sc_conv = functools.partial(
    pl.kernel,
    out_type=(jax.ShapeDtypeStruct((2, NPAD, D), _f32),
              jax.ShapeDtypeStruct((NTILE * NPAD,), _f32)),
    mesh=plsc.VectorSubcoreMesh(core_axis_name="c", subcore_axis_name="s",
                                num_cores=2, num_subcores=16),
    compiler_params=pltpu.CompilerParams(needs_layout_passes=False),
    scratch_types=(
        pltpu.VMEM((NPAD,), _f32),
        pltpu.VMEM((NPAD,), _f32),
        pltpu.VMEM((NPAD,), _f32),
        pltpu.VMEM((CH // 128, 128), _i32),
        pltpu.VMEM((CH // 128, 128), _i32),
        pltpu.VMEM((CH,), _f32),
        pltpu.VMEM((CH, D), _f32),
        pltpu.VMEM((16,), _f32),
        pltpu.VMEM_SHARED((NPAD, D), _f32),
    ),
)(_sc_conv_body)


def _erf(z):
    a = jnp.abs(z)
    t = 1.0 / (1.0 + 0.3275911 * a)
    poly = t * (0.254829592 + t * (-0.284496736 + t * (1.421413741
                + t * (-1.453152027 + t * 1.061405429))))
    return jnp.sign(z) * (1.0 - poly * jnp.exp(-a * a))


def _gelu(v):
    return 0.5 * v * (1.0 + _erf(v * 0.7071067811865476))


def _ln(v, g, b):
    mu = jnp.mean(v, axis=-1, keepdims=True)
    c = v - mu
    var = jnp.mean(c * c, axis=-1, keepdims=True)
    return c * lax.rsqrt(var + 1e-12) * g + b


def _dot(a, b):
    return jnp.dot(a, b, preferred_element_type=_f32)


def _conv_prep(x2, gw_ref, gas_ref, gad_ref, h_ref, as_ref, ad_ref,
               mas_ref, mad_ref, first):
    h = _dot(x2, gw_ref[...])
    h_ref[...] = h
    a_s = jnp.sum(h * gas_ref[...], axis=-1, keepdims=True)
    a_d = jnp.sum(h * gad_ref[...], axis=-1, keepdims=True)
    as_ref[...] = a_s
    ad_ref[...] = a_d

    @pl.when(first)
    def _():
        mas_ref[...] = jnp.full((1, 1), -1e30, _f32)
        mad_ref[...] = jnp.full((1, 1), -1e30, _f32)
    mas_ref[...] = jnp.maximum(mas_ref[...], jnp.max(a_s))
    mad_ref[...] = jnp.maximum(mad_ref[...], jnp.max(a_d))


def _stage_a_body(x_ref, wpre_ref, bpre_ref,
                  gw0_ref, gas0_ref, gad0_ref, gw1_ref, gas1_ref, gad1_ref,
                  x1_ref, h0_ref, as0_ref, ad0_ref, h1_ref, as1_ref, ad1_ref,
                  mas0_ref, mad0_ref, mas1_ref, mad1_ref):
    x = x_ref[...]
    t = _dot(x, wpre_ref[...]) + bpre_ref[...]
    x1 = jnp.maximum(t, 0.01 * t)
    x1_ref[...] = x1
    first = pl.program_id(0) == 0
    _conv_prep(x1, gw0_ref, gas0_ref, gad0_ref, h0_ref, as0_ref, ad0_ref,
               mas0_ref, mad0_ref, first)
    _conv_prep(x1, gw1_ref, gas1_ref, gad1_ref, h1_ref, as1_ref, ad1_ref,
               mas1_ref, mad1_ref, first)


def _stage_b_body(final, prex_ref, acc0_ref, den0_ref, acc1_ref, den1_ref,
                  gb0_ref, gb1_ref, wq_ref, bq_ref, wk_ref, bk_ref,
                  wv_ref, bv_ref, lng_ref, lnb_ref,
                  wm_ref, bm_ref, wo_ref, bo_ref, olg_ref, olb_ref,
                  *tail):
    prex = prex_ref[...]

    def _xt(acc_ref, den_ref, gb_ref):
        dsum = jnp.sum(den_ref[...], axis=-1, keepdims=True)
        return acc_ref[...] / dsum + gb_ref[...]

    xt0 = _xt(acc0_ref, den0_ref, gb0_ref)
    xt1 = _xt(acc1_ref, den1_ref, gb1_ref)

    q = _dot(prex, wq_ref[...]) + bq_ref[...]
    k0 = _dot(xt0, wk_ref[...]) + bk_ref[...]
    k1 = _dot(xt1, wk_ref[...]) + bk_ref[...]
    l0 = jnp.sum(q * k0, axis=-1, keepdims=True)
    l1 = jnp.sum(q * k1, axis=-1, keepdims=True)
    m = jnp.maximum(l0, l1)
    w0 = jnp.exp(l0 - m)
    w1 = jnp.exp(l1 - m)
    v0 = _dot(xt0, wv_ref[...]) + bv_ref[...]
    v1 = _dot(xt1, wv_ref[...]) + bv_ref[...]
    xatt = (w0 * v0 + w1 * v1) / (w0 + w1)

    x = _ln(prex + _gelu(xatt), lng_ref[...], lnb_ref[...])
    midv = _gelu(_dot(x, wm_ref[...]) + bm_ref[...])
    mid2 = _dot(midv, wo_ref[...]) + bo_ref[...]
    x2 = _ln(x + mid2, olg_ref[...], olb_ref[...])

    if final:
        wfin_ref, bfin_ref, out_ref = tail
        out_ref[...] = _gelu(_dot(x2, wfin_ref[...]) + bfin_ref[...])
    else:
        (gw0_ref, gas0_ref, gad0_ref, gw1_ref, gas1_ref, gad1_ref,
         x2_ref, h0_ref, as0_ref, ad0_ref, h1_ref, as1_ref, ad1_ref,
         mas0_ref, mad0_ref, mas1_ref, mad1_ref) = tail
        x2_ref[...] = x2
        first = pl.program_id(0) == 0
        _conv_prep(x2, gw0_ref, gas0_ref, gad0_ref, h0_ref, as0_ref, ad0_ref,
                   mas0_ref, mad0_ref, first)
        _conv_prep(x2, gw1_ref, gas1_ref, gad1_ref, h1_ref, as1_ref, ad1_ref,
                   mas1_ref, mad1_ref, first)


def _blk(shape, idx):
    return pl.BlockSpec(shape, idx)


_ROWB = _blk((R, D), lambda i: (i, 0))
_W128 = _blk((D, D), lambda i: (0, 0))
_ROW1 = _blk((1, D), lambda i: (0, 0))
_COL1 = _blk((R, 1), lambda i: (i, 0))
_SCLR = _blk((1, 1), lambda i: (0, 0))

_CONV_OUT_SHAPES = (
    jax.ShapeDtypeStruct((N, D), _f32),
    jax.ShapeDtypeStruct((N, 1), _f32),
    jax.ShapeDtypeStruct((N, 1), _f32),
)
_CONV_OUT_SPECS = (_ROWB, _COL1, _COL1)
_MAX_OUT = (jax.ShapeDtypeStruct((1, 1), _f32),) * 2
_MAX_SPEC = (_SCLR, _SCLR)


def _stage_a(x, wpre, bpre, gw0, gas0, gad0, gw1, gas1, gad1):
    return pl.pallas_call(
        _stage_a_body,
        grid=(GRID,),
        in_specs=[_ROWB, _W128, _ROW1,
                  _W128, _ROW1, _ROW1, _W128, _ROW1, _ROW1],
        out_specs=(_ROWB,) + _CONV_OUT_SPECS + _CONV_OUT_SPECS
        + _MAX_SPEC + _MAX_SPEC,
        out_shape=(jax.ShapeDtypeStruct((N, D), _f32),)
        + _CONV_OUT_SHAPES + _CONV_OUT_SHAPES + _MAX_OUT + _MAX_OUT,
    )(x, wpre, bpre, gw0, gas0, gad0, gw1, gas1, gad1)


_ACCB = _blk((R, D), lambda i: (i, 0))
_DENB = _blk((R, 16), lambda i: (i, 0))
_WMID = _blk((D, MID), lambda i: (0, 0))
_ROWM = _blk((1, MID), lambda i: (0, 0))
_WOUT = _blk((MID, D), lambda i: (0, 0))

_B_COMMON_SPECS = [
    _ROWB, _ACCB, _DENB, _ACCB, _DENB,
    _ROW1, _ROW1, _W128, _ROW1, _W128, _ROW1, _W128, _ROW1,
    _ROW1, _ROW1, _WMID, _ROWM, _WOUT, _ROW1, _ROW1, _ROW1,
]


def _stage_b_mid(args, tail_weights):
    return pl.pallas_call(
        functools.partial(_stage_b_body, False),
        grid=(GRID,),
        in_specs=_B_COMMON_SPECS + [_W128, _ROW1, _ROW1, _W128, _ROW1, _ROW1],
        out_specs=(_ROWB,) + _CONV_OUT_SPECS + _CONV_OUT_SPECS
        + _MAX_SPEC + _MAX_SPEC,
        out_shape=(jax.ShapeDtypeStruct((N, D), _f32),)
        + _CONV_OUT_SHAPES + _CONV_OUT_SHAPES + _MAX_OUT + _MAX_OUT,
    )(*args, *tail_weights)


def _stage_b_fin(args, tail_weights):
    return pl.pallas_call(
        functools.partial(_stage_b_body, True),
        grid=(GRID,),
        in_specs=_B_COMMON_SPECS + [_W128, _ROW1],
        out_specs=_ROWB,
        out_shape=jax.ShapeDtypeStruct((N, D), _f32),
    )(*args, *tail_weights)


def _edge_arrays(edge_index):
    ar = jnp.arange(N, dtype=_i32)
    pad_s = jnp.zeros((TOT - ETOT,), _i32)
    pad_d = jnp.full((TOT - ETOT,), N, _i32)
    srcs, dsts = [], []
    for j in range(2):
        src = jnp.concatenate([edge_index[j, 0].astype(_i32), ar, pad_s])
        dst = jnp.concatenate([edge_index[j, 1].astype(_i32), ar, pad_d])
        srcs.append(src.reshape(EROWS, 128))
        dsts.append(dst.reshape(EROWS, 128))
    return jnp.stack(srcs), jnp.stack(dsts)


def _table(a_s, a_d, mas, mad):
    c0 = mas[0, 0] + mad[0, 0]
    c = jnp.maximum(c0, 0.2 * c0)
    c16 = jnp.broadcast_to(c, (16,))
    asp = jnp.pad(a_s[:, 0], (0, NPAD - N))
    adp = jnp.pad(a_d[:, 0], (0, NPAD - N))
    return asp, adp, c16


def _run_convs(edges, cv0, cv1, h0, h1):
    as0, ad0, c0 = _table(*cv0)
    as1, ad1, c1 = _table(*cv1)
    acc, den = _sc_conv(edges[0], edges[1],
                        jnp.stack([as0, as1]), jnp.stack([ad0, ad1]),
                        jnp.stack([c0, c1]), jnp.stack([h0, h1]))
    den3 = den.reshape(2, 16, NPAD)
    return (acc[0, :N], den3[0, :, :N].T,
            acc[1, :N], den3[1, :, :N].T)


def kernel(x, params, edge_index):
    p = params

    def rowv(v):
        return v.reshape(1, -1).astype(_f32)

    edges = _edge_arrays(edge_index)

    def conv_w(i, j):
        return (p['gW_%d_%d' % (i, j)], rowv(p['gas_%d_%d' % (i, j)]),
                rowv(p['gad_%d_%d' % (i, j)]))

    (x1, h00, as00, ad00, h01, as01, ad01,
     mas00, mad00, mas01, mad01) = _stage_a(
        x, p['Wpre'], rowv(p['bpre']), *conv_w(0, 0), *conv_w(0, 1))

    acc00, den00, acc01, den01 = _run_convs(
        edges, (as00, ad00, mas00, mad00), (as01, ad01, mas01, mad01),
        h00, h01)

    def layer_args(i, prex, acc0, den0, acc1, den1):
        temp = p['atemp_%d' % i]
        return (prex, acc0, den0, acc1, den1,
                rowv(p['gb_%d_0' % i]), rowv(p['gb_%d_1' % i]),
                p['aWq_%d' % i] * temp, rowv(p['abq_%d' % i]) * temp,
                p['aWk_%d' % i], rowv(p['abk_%d' % i]),
                p['aWv_%d' % i], rowv(p['abv_%d' % i]),
                rowv(p['ln_g']), rowv(p['ln_b']),
                p['oWm_%d' % i], rowv(p['obm_%d' % i]),
                p['oWo_%d' % i], rowv(p['obo_%d' % i]),
                rowv(p['olg_%d' % i]), rowv(p['olb_%d' % i]))

    (x2, h10, as10, ad10, h11, as11, ad11,
     mas10, mad10, mas11, mad11) = _stage_b_mid(
        layer_args(0, x1, acc00, den00, acc01, den01),
        conv_w(1, 0) + conv_w(1, 1))

    acc10, den10, acc11, den11 = _run_convs(
        edges, (as10, ad10, mas10, mad10), (as11, ad11, mas11, mad11),
        h10, h11)

    return _stage_b_fin(
        layer_args(1, x2, acc10, den10, acc11, den11),
        (p['Wfin'], rowv(p['bfin'])))


# speedup vs baseline: 25.8652x; 1.1638x over previous
import functools

import jax
import jax.numpy as jnp
from jax import lax
from jax.experimental import pallas as pl
from jax.experimental.pallas import tpu as pltpu
from jax.experimental.pallas import tpu_sc as plsc

N = 10000
D = 128
MID = 512
E = 320000
ETOT = E + N
NTILE = 32
CH = 64
T_PER_TILE = 20736
NCHUNK = T_PER_TILE // CH
TOT = 16 * T_PER_TILE
EROWS = TOT // CH
NPAD = 10112
ZROWS = NPAD // 16

R = 400
GRID = N // R

_f32 = jnp.float32
_i32 = jnp.int32


def _sc_conv_body(src_hbm, dst_hbm, asrc_hbm, adst_hbm, c_hbm, h_hbm,
                  acc_out, den_out,
                  asrc_v, adst_v, den_v, src_b, dst_b, dsts_b, ee_b, rows_b,
                  c_v, acc_s, sem_is, sem_id, sem_g, sem_s):
    cid = lax.axis_index("c")
    sid = lax.axis_index("s")
    wid = cid * 16 + sid

    pltpu.sync_copy(asrc_hbm.at[cid], asrc_v)
    pltpu.sync_copy(adst_hbm.at[cid], adst_v)
    pltpu.sync_copy(c_hbm.at[cid], c_v)
    cvec = c_v[...]

    zf = jnp.zeros((16,), _f32)

    for b in range(2):
        def _zr(r, carry, _b=b):
            for f in range(8):
                rows_b[_b][r, pl.ds(f * 16, 16)] = zf
            return carry
        lax.fori_loop(0, CH, _zr, 0)

    def _zd(i, carry):
        den_v[pl.ds(pl.multiple_of(i * 16, 16), 16)] = zf
        return carry
    lax.fori_loop(0, NPAD // 16, _zd, 0)

    zbase = pl.multiple_of(sid * ZROWS, 8)
    nfull = ZROWS // CH
    for k in range(nfull):
        pltpu.sync_copy(rows_b[0], acc_s.at[pl.ds(zbase + k * CH, CH)])
    rem = ZROWS - nfull * CH
    if rem:
        pltpu.sync_copy(rows_b[0].at[pl.ds(0, rem)],
                        acc_s.at[pl.ds(zbase + nfull * CH, rem)])
    plsc.subcore_barrier()

    def _idx_copy(g, b):
        row = sid * NCHUNK + g
        pltpu.async_copy(src_hbm.at[cid, pl.ds(row, 1)], src_b[b], sem_is[b])
        pltpu.async_copy(dst_hbm.at[cid, pl.ds(row, 1)], dst_b[b], sem_id[b])

    def _idx_wait(g, b):
        row = sid * NCHUNK + g
        pltpu.make_async_copy(src_hbm.at[cid, pl.ds(row, 1)], src_b[b],
                              sem_is[b]).wait()
        pltpu.make_async_copy(dst_hbm.at[cid, pl.ds(row, 1)], dst_b[b],
                              sem_id[b]).wait()

    def _scatter_wait(b):
        pltpu.make_async_copy(rows_b[b], acc_s.at[dsts_b[b].at[0]],
                              sem_s[b]).wait()

    _idx_copy(0, 0)
    _idx_copy(1, 1)

    def _pair(go, carry):
        for b in range(2):
            g = go * 2 + b
            _idx_wait(g, b)

            @pl.when(go >= 1)
            def _():
                _scatter_wait(b)
            gdesc = pltpu.async_copy(h_hbm.at[cid].at[src_b[b].at[0]],
                                     rows_b[b], sem_g[b])

            def _sp(cc, carry2, _b=b):
                off = pl.multiple_of(cc * 16, 16)
                s16 = src_b[_b][0, pl.ds(off, 16)]
                d16 = dst_b[_b][0, pl.ds(off, 16)]
                dsts_b[_b][0, pl.ds(off, 16)] = d16
                av = plsc.load_gather(asrc_v, [s16])
                bv = plsc.load_gather(adst_v, [d16])
                e = av + bv
                e = jnp.maximum(e, 0.2 * e)
                ee = jnp.exp(e - cvec)
                ee_b[_b][pl.ds(off, 16)] = ee
                plsc.addupdate_scatter(den_v, [d16], ee)
                return carry2
            lax.fori_loop(0, CH // 16, _sp, 0)

            gdesc.wait()
            @pl.when(go < NCHUNK // 2 - 1)
            def _():
                _idx_copy(g + 2, b)

            def _scale(ei, carry2, _b=b):
                eev = plsc.load_gather(ee_b[_b], [jnp.zeros((16,), _i32) + ei])
                for f in range(8):
                    sl = pl.ds(f * 16, 16)
                    rows_b[_b][ei, sl] = rows_b[_b][ei, sl] * eev
                return carry2
            lax.fori_loop(0, CH, _scale, 0)

            pltpu.async_copy(rows_b[b], acc_s.at[dsts_b[b].at[0]], sem_s[b],
                             add=True)
        return carry
    lax.fori_loop(0, NCHUNK // 2, _pair, 0)
    _scatter_wait(0)
    _scatter_wait(1)

    plsc.subcore_barrier()
    obase = pl.multiple_of(sid * ZROWS, 8)
    pltpu.sync_copy(acc_s.at[pl.ds(obase, ZROWS)],
                    acc_out.at[cid, pl.ds(obase, ZROWS)])
    pltpu.sync_copy(den_v, den_out.at[pl.ds(pl.multiple_of(wid * NPAD, 128),
                                            NPAD)])


_sc_conv = functools.partial(
    pl.kernel,
    out_type=(jax.ShapeDtypeStruct((2, NPAD, D), _f32),
              jax.ShapeDtypeStruct((NTILE * NPAD,), _f32)),
    mesh=plsc.VectorSubcoreMesh(core_axis_name="c", subcore_axis_name="s",
                                num_cores=2, num_subcores=16),
    compiler_params=pltpu.CompilerParams(needs_layout_passes=False),
    scratch_types=(
        pltpu.VMEM((NPAD,), _f32),
        pltpu.VMEM((NPAD,), _f32),
        pltpu.VMEM((NPAD,), _f32),
        (pltpu.VMEM((1, CH), _i32),) * 2,
        (pltpu.VMEM((1, CH), _i32),) * 2,
        (pltpu.VMEM((1, CH), _i32),) * 2,
        (pltpu.VMEM((CH,), _f32),) * 2,
        (pltpu.VMEM((CH, D), _f32),) * 2,
        pltpu.VMEM((16,), _f32),
        pltpu.VMEM_SHARED((NPAD, D), _f32),
        (pltpu.SemaphoreType.DMA,) * 2,
        (pltpu.SemaphoreType.DMA,) * 2,
        (pltpu.SemaphoreType.DMA,) * 2,
        (pltpu.SemaphoreType.DMA,) * 2,
    ),
)(_sc_conv_body)


def _erf(z):
    a = jnp.abs(z)
    t = 1.0 / (1.0 + 0.3275911 * a)
    poly = t * (0.254829592 + t * (-0.284496736 + t * (1.421413741
                + t * (-1.453152027 + t * 1.061405429))))
    return jnp.sign(z) * (1.0 - poly * jnp.exp(-a * a))


def _gelu(v):
    return 0.5 * v * (1.0 + _erf(v * 0.7071067811865476))


def _ln(v, g, b):
    mu = jnp.mean(v, axis=-1, keepdims=True)
    c = v - mu
    var = jnp.mean(c * c, axis=-1, keepdims=True)
    return c * lax.rsqrt(var + 1e-12) * g + b


def _dot(a, b):
    return jnp.dot(a, b, preferred_element_type=_f32)


def _conv_prep(x2, gw_ref, gas_ref, gad_ref, h_ref, as_ref, ad_ref,
               mas_ref, mad_ref, first):
    h = _dot(x2, gw_ref[...])
    h_ref[...] = h
    a_s = jnp.sum(h * gas_ref[...], axis=-1, keepdims=True)
    a_d = jnp.sum(h * gad_ref[...], axis=-1, keepdims=True)
    as_ref[...] = a_s
    ad_ref[...] = a_d

    @pl.when(first)
    def _():
        mas_ref[...] = jnp.full((1, 1), -1e30, _f32)
        mad_ref[...] = jnp.full((1, 1), -1e30, _f32)
    mas_ref[...] = jnp.maximum(mas_ref[...], jnp.max(a_s))
    mad_ref[...] = jnp.maximum(mad_ref[...], jnp.max(a_d))


def _stage_a_body(x_ref, wpre_ref, bpre_ref,
                  gw0_ref, gas0_ref, gad0_ref, gw1_ref, gas1_ref, gad1_ref,
                  x1_ref, h0_ref, as0_ref, ad0_ref, h1_ref, as1_ref, ad1_ref,
                  mas0_ref, mad0_ref, mas1_ref, mad1_ref):
    x = x_ref[...]
    t = _dot(x, wpre_ref[...]) + bpre_ref[...]
    x1 = jnp.maximum(t, 0.01 * t)
    x1_ref[...] = x1
    first = pl.program_id(0) == 0
    _conv_prep(x1, gw0_ref, gas0_ref, gad0_ref, h0_ref, as0_ref, ad0_ref,
               mas0_ref, mad0_ref, first)
    _conv_prep(x1, gw1_ref, gas1_ref, gad1_ref, h1_ref, as1_ref, ad1_ref,
               mas1_ref, mad1_ref, first)


def _stage_b_body(final, prex_ref, acc0_ref, den0_ref, acc1_ref, den1_ref,
                  gb0_ref, gb1_ref, wq_ref, bq_ref, wk_ref, bk_ref,
                  wv_ref, bv_ref, lng_ref, lnb_ref,
                  wm_ref, bm_ref, wo_ref, bo_ref, olg_ref, olb_ref,
                  *tail):
    prex = prex_ref[...]

    def _xt(acc_ref, den_ref, gb_ref):
        dsum = jnp.sum(den_ref[...], axis=-1, keepdims=True)
        return acc_ref[...] / dsum + gb_ref[...]

    xt0 = _xt(acc0_ref, den0_ref, gb0_ref)
    xt1 = _xt(acc1_ref, den1_ref, gb1_ref)

    q = _dot(prex, wq_ref[...]) + bq_ref[...]
    k0 = _dot(xt0, wk_ref[...]) + bk_ref[...]
    k1 = _dot(xt1, wk_ref[...]) + bk_ref[...]
    l0 = jnp.sum(q * k0, axis=-1, keepdims=True)
    l1 = jnp.sum(q * k1, axis=-1, keepdims=True)
    m = jnp.maximum(l0, l1)
    w0 = jnp.exp(l0 - m)
    w1 = jnp.exp(l1 - m)
    v0 = _dot(xt0, wv_ref[...]) + bv_ref[...]
    v1 = _dot(xt1, wv_ref[...]) + bv_ref[...]
    xatt = (w0 * v0 + w1 * v1) / (w0 + w1)

    x = _ln(prex + _gelu(xatt), lng_ref[...], lnb_ref[...])
    midv = _gelu(_dot(x, wm_ref[...]) + bm_ref[...])
    mid2 = _dot(midv, wo_ref[...]) + bo_ref[...]
    x2 = _ln(x + mid2, olg_ref[...], olb_ref[...])

    if final:
        wfin_ref, bfin_ref, out_ref = tail
        out_ref[...] = _gelu(_dot(x2, wfin_ref[...]) + bfin_ref[...])
    else:
        (gw0_ref, gas0_ref, gad0_ref, gw1_ref, gas1_ref, gad1_ref,
         x2_ref, h0_ref, as0_ref, ad0_ref, h1_ref, as1_ref, ad1_ref,
         mas0_ref, mad0_ref, mas1_ref, mad1_ref) = tail
        x2_ref[...] = x2
        first = pl.program_id(0) == 0
        _conv_prep(x2, gw0_ref, gas0_ref, gad0_ref, h0_ref, as0_ref, ad0_ref,
                   mas0_ref, mad0_ref, first)
        _conv_prep(x2, gw1_ref, gas1_ref, gad1_ref, h1_ref, as1_ref, ad1_ref,
                   mas1_ref, mad1_ref, first)


def _blk(shape, idx):
    return pl.BlockSpec(shape, idx)


_ROWB = _blk((R, D), lambda i: (i, 0))
_W128 = _blk((D, D), lambda i: (0, 0))
_ROW1 = _blk((1, D), lambda i: (0, 0))
_COL1 = _blk((R, 1), lambda i: (i, 0))
_SCLR = _blk((1, 1), lambda i: (0, 0))

_CONV_OUT_SHAPES = (
    jax.ShapeDtypeStruct((N, D), _f32),
    jax.ShapeDtypeStruct((N, 1), _f32),
    jax.ShapeDtypeStruct((N, 1), _f32),
)
_CONV_OUT_SPECS = (_ROWB, _COL1, _COL1)
_MAX_OUT = (jax.ShapeDtypeStruct((1, 1), _f32),) * 2
_MAX_SPEC = (_SCLR, _SCLR)


def _stage_a(x, wpre, bpre, gw0, gas0, gad0, gw1, gas1, gad1):
    return pl.pallas_call(
        _stage_a_body,
        grid=(GRID,),
        in_specs=[_ROWB, _W128, _ROW1,
                  _W128, _ROW1, _ROW1, _W128, _ROW1, _ROW1],
        out_specs=(_ROWB,) + _CONV_OUT_SPECS + _CONV_OUT_SPECS
        + _MAX_SPEC + _MAX_SPEC,
        out_shape=(jax.ShapeDtypeStruct((N, D), _f32),)
        + _CONV_OUT_SHAPES + _CONV_OUT_SHAPES + _MAX_OUT + _MAX_OUT,
    )(x, wpre, bpre, gw0, gas0, gad0, gw1, gas1, gad1)


_ACCB = _blk((R, D), lambda i: (i, 0))
_DENB = _blk((R, 16), lambda i: (i, 0))
_WMID = _blk((D, MID), lambda i: (0, 0))
_ROWM = _blk((1, MID), lambda i: (0, 0))
_WOUT = _blk((MID, D), lambda i: (0, 0))

_B_COMMON_SPECS = [
    _ROWB, _ACCB, _DENB, _ACCB, _DENB,
    _ROW1, _ROW1, _W128, _ROW1, _W128, _ROW1, _W128, _ROW1,
    _ROW1, _ROW1, _WMID, _ROWM, _WOUT, _ROW1, _ROW1, _ROW1,
]


def _stage_b_mid(args, tail_weights):
    return pl.pallas_call(
        functools.partial(_stage_b_body, False),
        grid=(GRID,),
        in_specs=_B_COMMON_SPECS + [_W128, _ROW1, _ROW1, _W128, _ROW1, _ROW1],
        out_specs=(_ROWB,) + _CONV_OUT_SPECS + _CONV_OUT_SPECS
        + _MAX_SPEC + _MAX_SPEC,
        out_shape=(jax.ShapeDtypeStruct((N, D), _f32),)
        + _CONV_OUT_SHAPES + _CONV_OUT_SHAPES + _MAX_OUT + _MAX_OUT,
    )(*args, *tail_weights)


def _stage_b_fin(args, tail_weights):
    return pl.pallas_call(
        functools.partial(_stage_b_body, True),
        grid=(GRID,),
        in_specs=_B_COMMON_SPECS + [_W128, _ROW1],
        out_specs=_ROWB,
        out_shape=jax.ShapeDtypeStruct((N, D), _f32),
    )(*args, *tail_weights)


def _edge_arrays(edge_index):
    ar = jnp.arange(N, dtype=_i32)
    pad_s = jnp.zeros((TOT - ETOT,), _i32)
    pad_d = jnp.full((TOT - ETOT,), N, _i32)
    srcs, dsts = [], []
    for j in range(2):
        src = jnp.concatenate([edge_index[j, 0].astype(_i32), ar, pad_s])
        dst = jnp.concatenate([edge_index[j, 1].astype(_i32), ar, pad_d])
        srcs.append(src.reshape(EROWS, CH))
        dsts.append(dst.reshape(EROWS, CH))
    return jnp.stack(srcs), jnp.stack(dsts)


def _table(a_s, a_d, mas, mad):
    c0 = mas[0, 0] + mad[0, 0]
    c = jnp.maximum(c0, 0.2 * c0)
    c16 = jnp.broadcast_to(c, (16,))
    asp = jnp.pad(a_s[:, 0], (0, NPAD - N))
    adp = jnp.pad(a_d[:, 0], (0, NPAD - N))
    return asp, adp, c16


def _run_convs(edges, cv0, cv1, h0, h1):
    as0, ad0, c0 = _table(*cv0)
    as1, ad1, c1 = _table(*cv1)
    acc, den = _sc_conv(edges[0], edges[1],
                        jnp.stack([as0, as1]), jnp.stack([ad0, ad1]),
                        jnp.stack([c0, c1]), jnp.stack([h0, h1]))
    den3 = den.reshape(2, 16, NPAD)
    return (acc[0, :N], den3[0, :, :N].T,
            acc[1, :N], den3[1, :, :N].T)


def kernel(x, params, edge_index):
    p = params

    def rowv(v):
        return v.reshape(1, -1).astype(_f32)

    edges = _edge_arrays(edge_index)

    def conv_w(i, j):
        return (p['gW_%d_%d' % (i, j)], rowv(p['gas_%d_%d' % (i, j)]),
                rowv(p['gad_%d_%d' % (i, j)]))

    (x1, h00, as00, ad00, h01, as01, ad01,
     mas00, mad00, mas01, mad01) = _stage_a(
        x, p['Wpre'], rowv(p['bpre']), *conv_w(0, 0), *conv_w(0, 1))

    acc00, den00, acc01, den01 = _run_convs(
        edges, (as00, ad00, mas00, mad00), (as01, ad01, mas01, mad01),
        h00, h01)

    def layer_args(i, prex, acc0, den0, acc1, den1):
        temp = p['atemp_%d' % i]
        return (prex, acc0, den0, acc1, den1,
                rowv(p['gb_%d_0' % i]), rowv(p['gb_%d_1' % i]),
                p['aWq_%d' % i] * temp, rowv(p['abq_%d' % i]) * temp,
                p['aWk_%d' % i], rowv(p['abk_%d' % i]),
                p['aWv_%d' % i], rowv(p['abv_%d' % i]),
                rowv(p['ln_g']), rowv(p['ln_b']),
                p['oWm_%d' % i], rowv(p['obm_%d' % i]),
                p['oWo_%d' % i], rowv(p['obo_%d' % i]),
                rowv(p['olg_%d' % i]), rowv(p['olb_%d' % i]))

    (x2, h10, as10, ad10, h11, as11, ad11,
     mas10, mad10, mas11, mad11) = _stage_b_mid(
        layer_args(0, x1, acc00, den00, acc01, den01),
        conv_w(1, 0) + conv_w(1, 1))

    acc10, den10, acc11, den11 = _run_convs(
        edges, (as10, ad10, mas10, mad10), (as11, ad11, mas11, mad11),
        h10, h11)

    return _stage_b_fin(
        layer_args(1, x2, acc10, den10, acc11, den11),
        (p['Wfin'], rowv(p['bfin'])))


# speedup vs baseline: 29.7923x; 1.1518x over previous
import functools

import jax
import jax.numpy as jnp
import numpy as np
from jax import lax
from jax.experimental import pallas as pl
from jax.experimental.pallas import tpu as pltpu
from jax.experimental.pallas import tpu_sc as plsc

N = 10000
D = 128
MID = 512
E = 320000
ETOT = E + N
NTILE = 32
CH = 64
T_PER_TILE = 20736
NCHUNK = T_PER_TILE // CH
TOT = 16 * T_PER_TILE
EROWS = TOT // CH
NPAD = 10112
ZROWS = NPAD // 16

R = 400
GRID = N // R

_f32 = jnp.float32
_i32 = jnp.int32
_PIB = lax.GatherScatterMode.PROMISE_IN_BOUNDS
_LANE = [np.full((16,), j, np.int32) for j in range(16)]


def _sc_conv_body(src_hbm, dst_hbm, asrc_hbm, adst_hbm, c_hbm, h_hbm,
                  acc_out, den_out,
                  asrc_v, adst_v, den_v, src_b, dst_b, dsts_b, ee_b, rows_b,
                  c_v, acc_s, sem_is, sem_id, sem_g, sem_s):
    cid = lax.axis_index("c")
    sid = lax.axis_index("s")
    wid = cid * 16 + sid

    pltpu.sync_copy(asrc_hbm.at[cid], asrc_v)
    pltpu.sync_copy(adst_hbm.at[cid], adst_v)
    pltpu.sync_copy(c_hbm.at[cid], c_v)
    cvec = c_v[...]

    zf = jnp.zeros((16,), _f32)

    for b in range(2):
        def _zr(r, carry, _b=b):
            for f in range(8):
                rows_b[_b][r, pl.ds(f * 16, 16)] = zf
            return carry
        lax.fori_loop(0, CH, _zr, 0)

    def _zd(i, carry):
        den_v[pl.ds(pl.multiple_of(i * 16, 16), 16)] = zf
        return carry
    lax.fori_loop(0, NPAD // 16, _zd, 0)

    zbase = pl.multiple_of(sid * ZROWS, 8)
    nfull = ZROWS // CH
    for k in range(nfull):
        pltpu.sync_copy(rows_b[0], acc_s.at[pl.ds(zbase + k * CH, CH)])
    rem = ZROWS - nfull * CH
    if rem:
        pltpu.sync_copy(rows_b[0].at[pl.ds(0, rem)],
                        acc_s.at[pl.ds(zbase + nfull * CH, rem)])
    plsc.subcore_barrier()

    def _idx_copy(g, b):
        row = sid * NCHUNK + g
        pltpu.async_copy(src_hbm.at[cid, pl.ds(row, 1)], src_b[b], sem_is[b])
        pltpu.async_copy(dst_hbm.at[cid, pl.ds(row, 1)], dst_b[b], sem_id[b])

    def _idx_wait(g, b):
        row = sid * NCHUNK + g
        pltpu.make_async_copy(src_hbm.at[cid, pl.ds(row, 1)], src_b[b],
                              sem_is[b]).wait()
        pltpu.make_async_copy(dst_hbm.at[cid, pl.ds(row, 1)], dst_b[b],
                              sem_id[b]).wait()

    def _scatter_wait(b):
        pltpu.make_async_copy(rows_b[b], acc_s.at[dsts_b[b].at[0]],
                              sem_s[b]).wait()

    _idx_copy(0, 0)
    _idx_copy(1, 1)

    def _pair(go, carry):
        for b in range(2):
            g = go * 2 + b
            _idx_wait(g, b)

            @pl.when(go >= 1)
            def _():
                _scatter_wait(b)
            gdesc = pltpu.async_copy(h_hbm.at[cid].at[src_b[b].at[0]],
                                     rows_b[b], sem_g[b])

            def _sp(cc, carry2, _b=b):
                off = pl.multiple_of(cc * 16, 16)
                s16 = src_b[_b][0, pl.ds(off, 16)]
                d16 = dst_b[_b][0, pl.ds(off, 16)]
                dsts_b[_b][0, pl.ds(off, 16)] = d16
                av = plsc.load_gather(asrc_v, [s16])
                bv = plsc.load_gather(adst_v, [d16])
                e = av + bv
                e = jnp.maximum(e, 0.2 * e)
                ee = jnp.exp(e - cvec)
                ee_b[_b][pl.ds(off, 16)] = ee
                plsc.addupdate_scatter(den_v, [d16], ee)
                return carry2
            lax.fori_loop(0, CH // 16, _sp, 0)

            gdesc.wait()
            @pl.when(go < NCHUNK // 2 - 1)
            def _():
                _idx_copy(g + 2, b)

            def _scale(gj, carry2, _b=b):
                base = pl.multiple_of(gj * 16, 16)
                eev = ee_b[_b][pl.ds(base, 16)]
                for j in range(16):
                    bc = eev.at[jnp.zeros((16,), _i32) + j].get(mode=_PIB)
                    er = base + j
                    for f in range(8):
                        sl = pl.ds(f * 16, 16)
                        rows_b[_b][er, sl] = rows_b[_b][er, sl] * bc
                return carry2
            lax.fori_loop(0, CH // 16, _scale, 0)

            pltpu.async_copy(rows_b[b], acc_s.at[dsts_b[b].at[0]], sem_s[b],
                             add=True)
        return carry
    lax.fori_loop(0, NCHUNK // 2, _pair, 0)
    _scatter_wait(0)
    _scatter_wait(1)

    plsc.subcore_barrier()
    obase = pl.multiple_of(sid * ZROWS, 8)
    pltpu.sync_copy(acc_s.at[pl.ds(obase, ZROWS)],
                    acc_out.at[cid, pl.ds(obase, ZROWS)])
    pltpu.sync_copy(den_v, den_out.at[pl.ds(pl.multiple_of(wid * NPAD, 128),
                                            NPAD)])


_sc_conv = functools.partial(
    pl.kernel,
    out_type=(jax.ShapeDtypeStruct((2, NPAD, D), _f32),
              jax.ShapeDtypeStruct((NTILE * NPAD,), _f32)),
    mesh=plsc.VectorSubcoreMesh(core_axis_name="c", subcore_axis_name="s",
                                num_cores=2, num_subcores=16),
    compiler_params=pltpu.CompilerParams(needs_layout_passes=False),
    scratch_types=(
        pltpu.VMEM((NPAD,), _f32),
        pltpu.VMEM((NPAD,), _f32),
        pltpu.VMEM((NPAD,), _f32),
        (pltpu.VMEM((1, CH), _i32),) * 2,
        (pltpu.VMEM((1, CH), _i32),) * 2,
        (pltpu.VMEM((1, CH), _i32),) * 2,
        (pltpu.VMEM((CH,), _f32),) * 2,
        (pltpu.VMEM((CH, D), _f32),) * 2,
        pltpu.VMEM((16,), _f32),
        pltpu.VMEM_SHARED((NPAD, D), _f32),
        (pltpu.SemaphoreType.DMA,) * 2,
        (pltpu.SemaphoreType.DMA,) * 2,
        (pltpu.SemaphoreType.DMA,) * 2,
        (pltpu.SemaphoreType.DMA,) * 2,
    ),
)(_sc_conv_body)


def _erf(z):
    a = jnp.abs(z)
    t = 1.0 / (1.0 + 0.3275911 * a)
    poly = t * (0.254829592 + t * (-0.284496736 + t * (1.421413741
                + t * (-1.453152027 + t * 1.061405429))))
    return jnp.sign(z) * (1.0 - poly * jnp.exp(-a * a))


def _gelu(v):
    return 0.5 * v * (1.0 + _erf(v * 0.7071067811865476))


def _ln(v, g, b):
    mu = jnp.mean(v, axis=-1, keepdims=True)
    c = v - mu
    var = jnp.mean(c * c, axis=-1, keepdims=True)
    return c * lax.rsqrt(var + 1e-12) * g + b


def _dot(a, b):
    return jnp.dot(a, b, preferred_element_type=_f32)


def _conv_prep(x2, gw_ref, gas_ref, gad_ref, h_ref, as_ref, ad_ref,
               mas_ref, mad_ref, first):
    h = _dot(x2, gw_ref[...])
    h_ref[...] = h
    a_s = jnp.sum(h * gas_ref[...], axis=-1, keepdims=True)
    a_d = jnp.sum(h * gad_ref[...], axis=-1, keepdims=True)
    as_ref[...] = a_s
    ad_ref[...] = a_d

    @pl.when(first)
    def _():
        mas_ref[...] = jnp.full((1, 1), -1e30, _f32)
        mad_ref[...] = jnp.full((1, 1), -1e30, _f32)
    mas_ref[...] = jnp.maximum(mas_ref[...], jnp.max(a_s))
    mad_ref[...] = jnp.maximum(mad_ref[...], jnp.max(a_d))


def _stage_a_body(x_ref, wpre_ref, bpre_ref,
                  gw0_ref, gas0_ref, gad0_ref, gw1_ref, gas1_ref, gad1_ref,
                  x1_ref, h0_ref, as0_ref, ad0_ref, h1_ref, as1_ref, ad1_ref,
                  mas0_ref, mad0_ref, mas1_ref, mad1_ref):
    x = x_ref[...]
    t = _dot(x, wpre_ref[...]) + bpre_ref[...]
    x1 = jnp.maximum(t, 0.01 * t)
    x1_ref[...] = x1
    first = pl.program_id(0) == 0
    _conv_prep(x1, gw0_ref, gas0_ref, gad0_ref, h0_ref, as0_ref, ad0_ref,
               mas0_ref, mad0_ref, first)
    _conv_prep(x1, gw1_ref, gas1_ref, gad1_ref, h1_ref, as1_ref, ad1_ref,
               mas1_ref, mad1_ref, first)


def _stage_b_body(final, prex_ref, acc0_ref, den0_ref, acc1_ref, den1_ref,
                  gb0_ref, gb1_ref, wq_ref, bq_ref, wk_ref, bk_ref,
                  wv_ref, bv_ref, lng_ref, lnb_ref,
                  wm_ref, bm_ref, wo_ref, bo_ref, olg_ref, olb_ref,
                  *tail):
    prex = prex_ref[...]

    def _xt(acc_ref, den_ref, gb_ref):
        dsum = jnp.sum(den_ref[...], axis=-1, keepdims=True)
        return acc_ref[...] / dsum + gb_ref[...]

    xt0 = _xt(acc0_ref, den0_ref, gb0_ref)
    xt1 = _xt(acc1_ref, den1_ref, gb1_ref)

    q = _dot(prex, wq_ref[...]) + bq_ref[...]
    k0 = _dot(xt0, wk_ref[...]) + bk_ref[...]
    k1 = _dot(xt1, wk_ref[...]) + bk_ref[...]
    l0 = jnp.sum(q * k0, axis=-1, keepdims=True)
    l1 = jnp.sum(q * k1, axis=-1, keepdims=True)
    m = jnp.maximum(l0, l1)
    w0 = jnp.exp(l0 - m)
    w1 = jnp.exp(l1 - m)
    v0 = _dot(xt0, wv_ref[...]) + bv_ref[...]
    v1 = _dot(xt1, wv_ref[...]) + bv_ref[...]
    xatt = (w0 * v0 + w1 * v1) / (w0 + w1)

    x = _ln(prex + _gelu(xatt), lng_ref[...], lnb_ref[...])
    midv = _gelu(_dot(x, wm_ref[...]) + bm_ref[...])
    mid2 = _dot(midv, wo_ref[...]) + bo_ref[...]
    x2 = _ln(x + mid2, olg_ref[...], olb_ref[...])

    if final:
        wfin_ref, bfin_ref, out_ref = tail
        out_ref[...] = _gelu(_dot(x2, wfin_ref[...]) + bfin_ref[...])
    else:
        (gw0_ref, gas0_ref, gad0_ref, gw1_ref, gas1_ref, gad1_ref,
         x2_ref, h0_ref, as0_ref, ad0_ref, h1_ref, as1_ref, ad1_ref,
         mas0_ref, mad0_ref, mas1_ref, mad1_ref) = tail
        x2_ref[...] = x2
        first = pl.program_id(0) == 0
        _conv_prep(x2, gw0_ref, gas0_ref, gad0_ref, h0_ref, as0_ref, ad0_ref,
                   mas0_ref, mad0_ref, first)
        _conv_prep(x2, gw1_ref, gas1_ref, gad1_ref, h1_ref, as1_ref, ad1_ref,
                   mas1_ref, mad1_ref, first)


def _blk(shape, idx):
    return pl.BlockSpec(shape, idx)


_ROWB = _blk((R, D), lambda i: (i, 0))
_W128 = _blk((D, D), lambda i: (0, 0))
_ROW1 = _blk((1, D), lambda i: (0, 0))
_COL1 = _blk((R, 1), lambda i: (i, 0))
_SCLR = _blk((1, 1), lambda i: (0, 0))

_CONV_OUT_SHAPES = (
    jax.ShapeDtypeStruct((N, D), _f32),
    jax.ShapeDtypeStruct((N, 1), _f32),
    jax.ShapeDtypeStruct((N, 1), _f32),
)
_CONV_OUT_SPECS = (_ROWB, _COL1, _COL1)
_MAX_OUT = (jax.ShapeDtypeStruct((1, 1), _f32),) * 2
_MAX_SPEC = (_SCLR, _SCLR)


def _stage_a(x, wpre, bpre, gw0, gas0, gad0, gw1, gas1, gad1):
    return pl.pallas_call(
        _stage_a_body,
        grid=(GRID,),
        in_specs=[_ROWB, _W128, _ROW1,
                  _W128, _ROW1, _ROW1, _W128, _ROW1, _ROW1],
        out_specs=(_ROWB,) + _CONV_OUT_SPECS + _CONV_OUT_SPECS
        + _MAX_SPEC + _MAX_SPEC,
        out_shape=(jax.ShapeDtypeStruct((N, D), _f32),)
        + _CONV_OUT_SHAPES + _CONV_OUT_SHAPES + _MAX_OUT + _MAX_OUT,
    )(x, wpre, bpre, gw0, gas0, gad0, gw1, gas1, gad1)


_ACCB = _blk((R, D), lambda i: (i, 0))
_DENB = _blk((R, 16), lambda i: (i, 0))
_WMID = _blk((D, MID), lambda i: (0, 0))
_ROWM = _blk((1, MID), lambda i: (0, 0))
_WOUT = _blk((MID, D), lambda i: (0, 0))

_B_COMMON_SPECS = [
    _ROWB, _ACCB, _DENB, _ACCB, _DENB,
    _ROW1, _ROW1, _W128, _ROW1, _W128, _ROW1, _W128, _ROW1,
    _ROW1, _ROW1, _WMID, _ROWM, _WOUT, _ROW1, _ROW1, _ROW1,
]


def _stage_b_mid(args, tail_weights):
    return pl.pallas_call(
        functools.partial(_stage_b_body, False),
        grid=(GRID,),
        in_specs=_B_COMMON_SPECS + [_W128, _ROW1, _ROW1, _W128, _ROW1, _ROW1],
        out_specs=(_ROWB,) + _CONV_OUT_SPECS + _CONV_OUT_SPECS
        + _MAX_SPEC + _MAX_SPEC,
        out_shape=(jax.ShapeDtypeStruct((N, D), _f32),)
        + _CONV_OUT_SHAPES + _CONV_OUT_SHAPES + _MAX_OUT + _MAX_OUT,
    )(*args, *tail_weights)


def _stage_b_fin(args, tail_weights):
    return pl.pallas_call(
        functools.partial(_stage_b_body, True),
        grid=(GRID,),
        in_specs=_B_COMMON_SPECS + [_W128, _ROW1],
        out_specs=_ROWB,
        out_shape=jax.ShapeDtypeStruct((N, D), _f32),
    )(*args, *tail_weights)


def _edge_arrays(edge_index):
    ar = jnp.arange(N, dtype=_i32)
    pad_s = jnp.zeros((TOT - ETOT,), _i32)
    pad_d = jnp.full((TOT - ETOT,), N, _i32)
    srcs, dsts = [], []
    for j in range(2):
        src = jnp.concatenate([edge_index[j, 0].astype(_i32), ar, pad_s])
        dst = jnp.concatenate([edge_index[j, 1].astype(_i32), ar, pad_d])
        srcs.append(src.reshape(EROWS, CH))
        dsts.append(dst.reshape(EROWS, CH))
    return jnp.stack(srcs), jnp.stack(dsts)


def _table(a_s, a_d, mas, mad):
    c0 = mas[0, 0] + mad[0, 0]
    c = jnp.maximum(c0, 0.2 * c0)
    c16 = jnp.broadcast_to(c, (16,))
    asp = jnp.pad(a_s[:, 0], (0, NPAD - N))
    adp = jnp.pad(a_d[:, 0], (0, NPAD - N))
    return asp, adp, c16


def _run_convs(edges, cv0, cv1, h0, h1):
    as0, ad0, c0 = _table(*cv0)
    as1, ad1, c1 = _table(*cv1)
    acc, den = _sc_conv(edges[0], edges[1],
                        jnp.stack([as0, as1]), jnp.stack([ad0, ad1]),
                        jnp.stack([c0, c1]), jnp.stack([h0, h1]))
    den3 = den.reshape(2, 16, NPAD)
    return (acc[0, :N], den3[0, :, :N].T,
            acc[1, :N], den3[1, :, :N].T)


def kernel(x, params, edge_index):
    p = params

    def rowv(v):
        return v.reshape(1, -1).astype(_f32)

    edges = _edge_arrays(edge_index)

    def conv_w(i, j):
        return (p['gW_%d_%d' % (i, j)], rowv(p['gas_%d_%d' % (i, j)]),
                rowv(p['gad_%d_%d' % (i, j)]))

    (x1, h00, as00, ad00, h01, as01, ad01,
     mas00, mad00, mas01, mad01) = _stage_a(
        x, p['Wpre'], rowv(p['bpre']), *conv_w(0, 0), *conv_w(0, 1))

    acc00, den00, acc01, den01 = _run_convs(
        edges, (as00, ad00, mas00, mad00), (as01, ad01, mas01, mad01),
        h00, h01)

    def layer_args(i, prex, acc0, den0, acc1, den1):
        temp = p['atemp_%d' % i]
        return (prex, acc0, den0, acc1, den1,
                rowv(p['gb_%d_0' % i]), rowv(p['gb_%d_1' % i]),
                p['aWq_%d' % i] * temp, rowv(p['abq_%d' % i]) * temp,
                p['aWk_%d' % i], rowv(p['abk_%d' % i]),
                p['aWv_%d' % i], rowv(p['abv_%d' % i]),
                rowv(p['ln_g']), rowv(p['ln_b']),
                p['oWm_%d' % i], rowv(p['obm_%d' % i]),
                p['oWo_%d' % i], rowv(p['obo_%d' % i]),
                rowv(p['olg_%d' % i]), rowv(p['olb_%d' % i]))

    (x2, h10, as10, ad10, h11, as11, ad11,
     mas10, mad10, mas11, mad11) = _stage_b_mid(
        layer_args(0, x1, acc00, den00, acc01, den01),
        conv_w(1, 0) + conv_w(1, 1))

    acc10, den10, acc11, den11 = _run_convs(
        edges, (as10, ad10, mas10, mad10), (as11, ad11, mas11, mad11),
        h10, h11)

    return _stage_b_fin(
        layer_args(1, x2, acc10, den10, acc11, den11),
        (p['Wfin'], rowv(p['bfin'])))


# speedup vs baseline: 38.0812x; 1.2782x over previous
import functools

import jax
import jax.numpy as jnp
import numpy as np
from jax import lax
from jax.experimental import pallas as pl
from jax.experimental.pallas import tpu as pltpu
from jax.experimental.pallas import tpu_sc as plsc

N = 10000
D = 128
MID = 512
E = 320000
ETOT = E + N
NTILE = 32
CH = 64
T_PER_TILE = 20736
NCHUNK = T_PER_TILE // CH
TOT = 16 * T_PER_TILE
EROWS = TOT // CH
NPAD = 10112
ZROWS = NPAD // 16

R = 400
GRID = N // R

_f32 = jnp.float32
_i32 = jnp.int32
_PIB = lax.GatherScatterMode.PROMISE_IN_BOUNDS
_LANE = [np.full((16,), j, np.int32) for j in range(16)]


def _sc_conv_body(src_hbm, dst_hbm, asrc_hbm, adst_hbm, c_hbm, h_hbm,
                  acc_out, den_out,
                  asrc_v, adst_v, zbuf, src_b, dst_b, dsts_b, ee_b, rows_b,
                  c_v, acc_s, den_s, sem_is, sem_id, sem_g, sem_s, sem_d):
    cid = lax.axis_index("c")
    sid = lax.axis_index("s")

    pltpu.sync_copy(asrc_hbm.at[cid], asrc_v)
    pltpu.sync_copy(adst_hbm.at[cid], adst_v)
    pltpu.sync_copy(c_hbm.at[cid], c_v)
    cvec = c_v[...]

    zf = jnp.zeros((16,), _f32)

    def _zr(r, carry):
        for f in range(8):
            rows_b[0][r, pl.ds(f * 16, 16)] = zf
        return carry
    lax.fori_loop(0, CH, _zr, 0)

    def _zb(i, carry):
        zbuf[pl.ds(pl.multiple_of(i * 16, 16), 16)] = zf
        return carry
    lax.fori_loop(0, 640 // 16, _zb, 0)

    zbase = pl.multiple_of(sid * ZROWS, 8)
    nfull = ZROWS // CH
    for k in range(nfull):
        pltpu.sync_copy(rows_b[0], acc_s.at[pl.ds(zbase + k * CH, CH)])
    rem = ZROWS - nfull * CH
    if rem:
        pltpu.sync_copy(rows_b[0].at[pl.ds(0, rem)],
                        acc_s.at[pl.ds(zbase + nfull * CH, rem)])

    @pl.when(sid < 15)
    def _():
        pltpu.sync_copy(zbuf, den_s.at[pl.ds(pl.multiple_of(sid * 640, 8),
                                             640)])

    @pl.when(sid == 15)
    def _():
        pltpu.sync_copy(zbuf.at[pl.ds(0, NPAD - 15 * 640)],
                        den_s.at[pl.ds(15 * 640, NPAD - 15 * 640)])
    plsc.subcore_barrier()

    def _idx_copy(g, b):
        row = sid * NCHUNK + g
        pltpu.async_copy(src_hbm.at[cid, pl.ds(row, 1)], src_b[b], sem_is[b])
        pltpu.async_copy(dst_hbm.at[cid, pl.ds(row, 1)], dst_b[b], sem_id[b])

    def _idx_wait(g, b):
        row = sid * NCHUNK + g
        pltpu.make_async_copy(src_hbm.at[cid, pl.ds(row, 1)], src_b[b],
                              sem_is[b]).wait()
        pltpu.make_async_copy(dst_hbm.at[cid, pl.ds(row, 1)], dst_b[b],
                              sem_id[b]).wait()

    def _gather_issue(b):
        pltpu.async_copy(h_hbm.at[cid].at[src_b[b].at[0]], rows_b[b],
                         sem_g[b])

    def _gather_wait(b):
        pltpu.make_async_copy(h_hbm.at[cid].at[src_b[b].at[0]], rows_b[b],
                              sem_g[b]).wait()

    def _out_wait(b):
        pltpu.make_async_copy(rows_b[b], acc_s.at[dsts_b[b].at[0]],
                              sem_s[b]).wait()
        pltpu.make_async_copy(ee_b[b], den_s.at[dsts_b[b].at[0]],
                              sem_d[b]).wait()

    _idx_copy(0, 0)
    _idx_wait(0, 0)
    _gather_issue(0)
    _idx_copy(1, 1)

    def _trip(go, carry):
        for b in range(3):
            g = go * 3 + b
            bn = (b + 1) % 3

            @pl.when(g + 1 < NCHUNK)
            def _():
                _idx_wait(g + 1, bn)

                @pl.when(g >= 2)
                def _():
                    _out_wait(bn)
                _gather_issue(bn)

            _gather_wait(b)

            def _sp(cc, carry2, _b=b):
                off = pl.multiple_of(cc * 16, 16)
                s16 = src_b[_b][0, pl.ds(off, 16)]
                d16 = dst_b[_b][0, pl.ds(off, 16)]
                dsts_b[_b][0, pl.ds(off, 16)] = d16
                av = plsc.load_gather(asrc_v, [s16])
                bv = plsc.load_gather(adst_v, [d16])
                e = av + bv
                e = jnp.maximum(e, 0.2 * e)
                ee_b[_b][pl.ds(off, 16)] = jnp.exp(e - cvec)
                return carry2
            lax.fori_loop(0, CH // 16, _sp, 0)

            @pl.when(g + 2 < NCHUNK)
            def _():
                _idx_copy(g + 2, (b + 2) % 3)

            def _scale(gj, carry2, _b=b):
                base = pl.multiple_of(gj * 16, 16)
                eev = ee_b[_b][pl.ds(base, 16)]
                for j in range(16):
                    bc = eev.at[jnp.zeros((16,), _i32) + j].get(mode=_PIB)
                    er = base + j
                    for f in range(8):
                        sl = pl.ds(f * 16, 16)
                        rows_b[_b][er, sl] = rows_b[_b][er, sl] * bc
                return carry2
            lax.fori_loop(0, CH // 16, _scale, 0)

            pltpu.async_copy(rows_b[b], acc_s.at[dsts_b[b].at[0]], sem_s[b],
                             add=True)
            pltpu.async_copy(ee_b[b], den_s.at[dsts_b[b].at[0]], sem_d[b],
                             add=True)
        return carry
    lax.fori_loop(0, NCHUNK // 3, _trip, 0)
    for b in range(3):
        _out_wait(b)

    plsc.subcore_barrier()
    obase = pl.multiple_of(sid * ZROWS, 8)
    pltpu.sync_copy(acc_s.at[pl.ds(obase, ZROWS)],
                    acc_out.at[cid, pl.ds(obase, ZROWS)])

    @pl.when(sid == 0)
    def _():
        pltpu.sync_copy(den_s,
                        den_out.at[pl.ds(pl.multiple_of(cid * NPAD, 128),
                                         NPAD)])


_sc_conv = functools.partial(
    pl.kernel,
    out_type=(jax.ShapeDtypeStruct((2, NPAD, D), _f32),
              jax.ShapeDtypeStruct((2 * NPAD,), _f32)),
    mesh=plsc.VectorSubcoreMesh(core_axis_name="c", subcore_axis_name="s",
                                num_cores=2, num_subcores=16),
    compiler_params=pltpu.CompilerParams(needs_layout_passes=False),
    scratch_types=(
        pltpu.VMEM((NPAD,), _f32),
        pltpu.VMEM((NPAD,), _f32),
        pltpu.VMEM((640,), _f32),
        (pltpu.VMEM((1, CH), _i32),) * 3,
        (pltpu.VMEM((1, CH), _i32),) * 3,
        (pltpu.VMEM((1, CH), _i32),) * 3,
        (pltpu.VMEM((CH,), _f32),) * 3,
        (pltpu.VMEM((CH, D), _f32),) * 3,
        pltpu.VMEM((16,), _f32),
        pltpu.VMEM_SHARED((NPAD, D), _f32),
        pltpu.VMEM_SHARED((NPAD,), _f32),
        (pltpu.SemaphoreType.DMA,) * 3,
        (pltpu.SemaphoreType.DMA,) * 3,
        (pltpu.SemaphoreType.DMA,) * 3,
        (pltpu.SemaphoreType.DMA,) * 3,
        (pltpu.SemaphoreType.DMA,) * 3,
    ),
)(_sc_conv_body)


def _erf(z):
    a = jnp.abs(z)
    t = 1.0 / (1.0 + 0.3275911 * a)
    poly = t * (0.254829592 + t * (-0.284496736 + t * (1.421413741
                + t * (-1.453152027 + t * 1.061405429))))
    return jnp.sign(z) * (1.0 - poly * jnp.exp(-a * a))


def _gelu(v):
    return 0.5 * v * (1.0 + _erf(v * 0.7071067811865476))


def _ln(v, g, b):
    mu = jnp.mean(v, axis=-1, keepdims=True)
    c = v - mu
    var = jnp.mean(c * c, axis=-1, keepdims=True)
    return c * lax.rsqrt(var + 1e-12) * g + b


def _dot(a, b):
    return jnp.dot(a, b, preferred_element_type=_f32)


def _conv_prep(x2, gw_ref, gas_ref, gad_ref, h_ref, as_ref, ad_ref,
               mas_ref, mad_ref, first):
    h = _dot(x2, gw_ref[...])
    h_ref[...] = h
    a_s = jnp.sum(h * gas_ref[...], axis=-1, keepdims=True)
    a_d = jnp.sum(h * gad_ref[...], axis=-1, keepdims=True)
    as_ref[...] = a_s
    ad_ref[...] = a_d

    @pl.when(first)
    def _():
        mas_ref[...] = jnp.full((1, 1), -1e30, _f32)
        mad_ref[...] = jnp.full((1, 1), -1e30, _f32)
    mas_ref[...] = jnp.maximum(mas_ref[...], jnp.max(a_s))
    mad_ref[...] = jnp.maximum(mad_ref[...], jnp.max(a_d))


def _stage_a_body(x_ref, wpre_ref, bpre_ref,
                  gw0_ref, gas0_ref, gad0_ref, gw1_ref, gas1_ref, gad1_ref,
                  x1_ref, h0_ref, as0_ref, ad0_ref, h1_ref, as1_ref, ad1_ref,
                  mas0_ref, mad0_ref, mas1_ref, mad1_ref):
    x = x_ref[...]
    t = _dot(x, wpre_ref[...]) + bpre_ref[...]
    x1 = jnp.maximum(t, 0.01 * t)
    x1_ref[...] = x1
    first = pl.program_id(0) == 0
    _conv_prep(x1, gw0_ref, gas0_ref, gad0_ref, h0_ref, as0_ref, ad0_ref,
               mas0_ref, mad0_ref, first)
    _conv_prep(x1, gw1_ref, gas1_ref, gad1_ref, h1_ref, as1_ref, ad1_ref,
               mas1_ref, mad1_ref, first)


def _stage_b_body(final, prex_ref, acc0_ref, den0_ref, acc1_ref, den1_ref,
                  gb0_ref, gb1_ref, wq_ref, bq_ref, wk_ref, bk_ref,
                  wv_ref, bv_ref, lng_ref, lnb_ref,
                  wm_ref, bm_ref, wo_ref, bo_ref, olg_ref, olb_ref,
                  *tail):
    prex = prex_ref[...]

    def _xt(acc_ref, den_ref, gb_ref):
        return acc_ref[...] / den_ref[...] + gb_ref[...]

    xt0 = _xt(acc0_ref, den0_ref, gb0_ref)
    xt1 = _xt(acc1_ref, den1_ref, gb1_ref)

    q = _dot(prex, wq_ref[...]) + bq_ref[...]
    k0 = _dot(xt0, wk_ref[...]) + bk_ref[...]
    k1 = _dot(xt1, wk_ref[...]) + bk_ref[...]
    l0 = jnp.sum(q * k0, axis=-1, keepdims=True)
    l1 = jnp.sum(q * k1, axis=-1, keepdims=True)
    m = jnp.maximum(l0, l1)
    w0 = jnp.exp(l0 - m)
    w1 = jnp.exp(l1 - m)
    v0 = _dot(xt0, wv_ref[...]) + bv_ref[...]
    v1 = _dot(xt1, wv_ref[...]) + bv_ref[...]
    xatt = (w0 * v0 + w1 * v1) / (w0 + w1)

    x = _ln(prex + _gelu(xatt), lng_ref[...], lnb_ref[...])
    midv = _gelu(_dot(x, wm_ref[...]) + bm_ref[...])
    mid2 = _dot(midv, wo_ref[...]) + bo_ref[...]
    x2 = _ln(x + mid2, olg_ref[...], olb_ref[...])

    if final:
        wfin_ref, bfin_ref, out_ref = tail
        out_ref[...] = _gelu(_dot(x2, wfin_ref[...]) + bfin_ref[...])
    else:
        (gw0_ref, gas0_ref, gad0_ref, gw1_ref, gas1_ref, gad1_ref,
         x2_ref, h0_ref, as0_ref, ad0_ref, h1_ref, as1_ref, ad1_ref,
         mas0_ref, mad0_ref, mas1_ref, mad1_ref) = tail
        x2_ref[...] = x2
        first = pl.program_id(0) == 0
        _conv_prep(x2, gw0_ref, gas0_ref, gad0_ref, h0_ref, as0_ref, ad0_ref,
                   mas0_ref, mad0_ref, first)
        _conv_prep(x2, gw1_ref, gas1_ref, gad1_ref, h1_ref, as1_ref, ad1_ref,
                   mas1_ref, mad1_ref, first)


def _blk(shape, idx):
    return pl.BlockSpec(shape, idx)


_ROWB = _blk((R, D), lambda i: (i, 0))
_W128 = _blk((D, D), lambda i: (0, 0))
_ROW1 = _blk((1, D), lambda i: (0, 0))
_COL1 = _blk((R, 1), lambda i: (i, 0))
_SCLR = _blk((1, 1), lambda i: (0, 0))

_CONV_OUT_SHAPES = (
    jax.ShapeDtypeStruct((N, D), _f32),
    jax.ShapeDtypeStruct((N, 1), _f32),
    jax.ShapeDtypeStruct((N, 1), _f32),
)
_CONV_OUT_SPECS = (_ROWB, _COL1, _COL1)
_MAX_OUT = (jax.ShapeDtypeStruct((1, 1), _f32),) * 2
_MAX_SPEC = (_SCLR, _SCLR)


def _stage_a(x, wpre, bpre, gw0, gas0, gad0, gw1, gas1, gad1):
    return pl.pallas_call(
        _stage_a_body,
        grid=(GRID,),
        in_specs=[_ROWB, _W128, _ROW1,
                  _W128, _ROW1, _ROW1, _W128, _ROW1, _ROW1],
        out_specs=(_ROWB,) + _CONV_OUT_SPECS + _CONV_OUT_SPECS
        + _MAX_SPEC + _MAX_SPEC,
        out_shape=(jax.ShapeDtypeStruct((N, D), _f32),)
        + _CONV_OUT_SHAPES + _CONV_OUT_SHAPES + _MAX_OUT + _MAX_OUT,
    )(x, wpre, bpre, gw0, gas0, gad0, gw1, gas1, gad1)


_ACCB = _blk((R, D), lambda i: (i, 0))
_DENB = _blk((R, 1), lambda i: (i, 0))
_WMID = _blk((D, MID), lambda i: (0, 0))
_ROWM = _blk((1, MID), lambda i: (0, 0))
_WOUT = _blk((MID, D), lambda i: (0, 0))

_B_COMMON_SPECS = [
    _ROWB, _ACCB, _DENB, _ACCB, _DENB,
    _ROW1, _ROW1, _W128, _ROW1, _W128, _ROW1, _W128, _ROW1,
    _ROW1, _ROW1, _WMID, _ROWM, _WOUT, _ROW1, _ROW1, _ROW1,
]


def _stage_b_mid(args, tail_weights):
    return pl.pallas_call(
        functools.partial(_stage_b_body, False),
        grid=(GRID,),
        in_specs=_B_COMMON_SPECS + [_W128, _ROW1, _ROW1, _W128, _ROW1, _ROW1],
        out_specs=(_ROWB,) + _CONV_OUT_SPECS + _CONV_OUT_SPECS
        + _MAX_SPEC + _MAX_SPEC,
        out_shape=(jax.ShapeDtypeStruct((N, D), _f32),)
        + _CONV_OUT_SHAPES + _CONV_OUT_SHAPES + _MAX_OUT + _MAX_OUT,
    )(*args, *tail_weights)


def _stage_b_fin(args, tail_weights):
    return pl.pallas_call(
        functools.partial(_stage_b_body, True),
        grid=(GRID,),
        in_specs=_B_COMMON_SPECS + [_W128, _ROW1],
        out_specs=_ROWB,
        out_shape=jax.ShapeDtypeStruct((N, D), _f32),
    )(*args, *tail_weights)


def _edge_arrays(edge_index):
    ar = jnp.arange(N, dtype=_i32)
    pad_s = jnp.zeros((TOT - ETOT,), _i32)
    pad_d = jnp.full((TOT - ETOT,), N, _i32)
    srcs, dsts = [], []
    for j in range(2):
        src = jnp.concatenate([edge_index[j, 0].astype(_i32), ar, pad_s])
        dst = jnp.concatenate([edge_index[j, 1].astype(_i32), ar, pad_d])
        srcs.append(src.reshape(EROWS, CH))
        dsts.append(dst.reshape(EROWS, CH))
    return jnp.stack(srcs), jnp.stack(dsts)


def _table(a_s, a_d, mas, mad):
    c0 = mas[0, 0] + mad[0, 0]
    c = jnp.maximum(c0, 0.2 * c0)
    c16 = jnp.broadcast_to(c, (16,))
    asp = jnp.pad(a_s[:, 0], (0, NPAD - N))
    adp = jnp.pad(a_d[:, 0], (0, NPAD - N))
    return asp, adp, c16


def _run_convs(edges, cv0, cv1, h0, h1):
    as0, ad0, c0 = _table(*cv0)
    as1, ad1, c1 = _table(*cv1)
    acc, den = _sc_conv(edges[0], edges[1],
                        jnp.stack([as0, as1]), jnp.stack([ad0, ad1]),
                        jnp.stack([c0, c1]), jnp.stack([h0, h1]))
    den2 = den.reshape(2, NPAD)
    return (acc[0, :N], den2[0, :N].reshape(N, 1),
            acc[1, :N], den2[1, :N].reshape(N, 1))


def kernel(x, params, edge_index):
    p = params

    def rowv(v):
        return v.reshape(1, -1).astype(_f32)

    edges = _edge_arrays(edge_index)

    def conv_w(i, j):
        return (p['gW_%d_%d' % (i, j)], rowv(p['gas_%d_%d' % (i, j)]),
                rowv(p['gad_%d_%d' % (i, j)]))

    (x1, h00, as00, ad00, h01, as01, ad01,
     mas00, mad00, mas01, mad01) = _stage_a(
        x, p['Wpre'], rowv(p['bpre']), *conv_w(0, 0), *conv_w(0, 1))

    acc00, den00, acc01, den01 = _run_convs(
        edges, (as00, ad00, mas00, mad00), (as01, ad01, mas01, mad01),
        h00, h01)

    def layer_args(i, prex, acc0, den0, acc1, den1):
        temp = p['atemp_%d' % i]
        return (prex, acc0, den0, acc1, den1,
                rowv(p['gb_%d_0' % i]), rowv(p['gb_%d_1' % i]),
                p['aWq_%d' % i] * temp, rowv(p['abq_%d' % i]) * temp,
                p['aWk_%d' % i], rowv(p['abk_%d' % i]),
                p['aWv_%d' % i], rowv(p['abv_%d' % i]),
                rowv(p['ln_g']), rowv(p['ln_b']),
                p['oWm_%d' % i], rowv(p['obm_%d' % i]),
                p['oWo_%d' % i], rowv(p['obo_%d' % i]),
                rowv(p['olg_%d' % i]), rowv(p['olb_%d' % i]))

    (x2, h10, as10, ad10, h11, as11, ad11,
     mas10, mad10, mas11, mad11) = _stage_b_mid(
        layer_args(0, x1, acc00, den00, acc01, den01),
        conv_w(1, 0) + conv_w(1, 1))

    acc10, den10, acc11, den11 = _run_convs(
        edges, (as10, ad10, mas10, mad10), (as11, ad11, mas11, mad11),
        h10, h11)

    return _stage_b_fin(
        layer_args(1, x2, acc10, den10, acc11, den11),
        (p['Wfin'], rowv(p['bfin'])))


# speedup vs baseline: 39.7079x; 1.0427x over previous
import functools

import jax
import jax.numpy as jnp
import numpy as np
from jax import lax
from jax.experimental import pallas as pl
from jax.experimental.pallas import tpu as pltpu
from jax.experimental.pallas import tpu_sc as plsc

N = 10000
D = 128
MID = 512
E = 320000
ETOT = E + N
NTILE = 32
CH = 64
T_PER_TILE = 20736
NCHUNK = T_PER_TILE // CH
TOT = 16 * T_PER_TILE
EROWS = TOT // CH
NPAD = 10112
ZROWS = NPAD // 16

R = 400
GRID = N // R

_f32 = jnp.float32
_i32 = jnp.int32
_PIB = lax.GatherScatterMode.PROMISE_IN_BOUNDS
_LANE = [np.full((16,), j, np.int32) for j in range(16)]


def _sc_conv_body(src_hbm, dst_hbm, asrc_hbm, adst_hbm, c_hbm, h_hbm,
                  acc_out, den_out,
                  asrc_v, adst_v, zbuf, src_b, dst_b, dsts_b, ee_b, rows_b,
                  c_v, acc_s, den_s, sem_is, sem_id, sem_g, sem_s, sem_d):
    cid = lax.axis_index("c")
    sid = lax.axis_index("s")

    pltpu.sync_copy(asrc_hbm.at[cid], asrc_v)
    pltpu.sync_copy(adst_hbm.at[cid], adst_v)
    pltpu.sync_copy(c_hbm.at[cid], c_v)
    cvec = c_v[...]

    zf = jnp.zeros((16,), _f32)

    def _zr(r, carry):
        for f in range(8):
            rows_b[0][r, pl.ds(f * 16, 16)] = zf
        return carry
    lax.fori_loop(0, CH, _zr, 0)

    def _zb(i, carry):
        zbuf[pl.ds(pl.multiple_of(i * 16, 16), 16)] = zf
        return carry
    lax.fori_loop(0, 640 // 16, _zb, 0)

    zbase = pl.multiple_of(sid * ZROWS, 8)
    nfull = ZROWS // CH
    for k in range(nfull):
        pltpu.sync_copy(rows_b[0], acc_s.at[pl.ds(zbase + k * CH, CH)])
    rem = ZROWS - nfull * CH
    if rem:
        pltpu.sync_copy(rows_b[0].at[pl.ds(0, rem)],
                        acc_s.at[pl.ds(zbase + nfull * CH, rem)])

    @pl.when(sid < 15)
    def _():
        pltpu.sync_copy(zbuf, den_s.at[pl.ds(pl.multiple_of(sid * 640, 8),
                                             640)])

    @pl.when(sid == 15)
    def _():
        pltpu.sync_copy(zbuf.at[pl.ds(0, NPAD - 15 * 640)],
                        den_s.at[pl.ds(15 * 640, NPAD - 15 * 640)])
    plsc.subcore_barrier()

    def _idx_copy(g, b):
        row = sid * NCHUNK + g
        pltpu.async_copy(src_hbm.at[cid, pl.ds(row, 1)], src_b[b], sem_is[b])
        pltpu.async_copy(dst_hbm.at[cid, pl.ds(row, 1)], dst_b[b], sem_id[b])

    def _idx_wait(g, b):
        row = sid * NCHUNK + g
        pltpu.make_async_copy(src_hbm.at[cid, pl.ds(row, 1)], src_b[b],
                              sem_is[b]).wait()
        pltpu.make_async_copy(dst_hbm.at[cid, pl.ds(row, 1)], dst_b[b],
                              sem_id[b]).wait()

    def _gather_issue(b):
        pltpu.async_copy(h_hbm.at[cid].at[src_b[b].at[0]], rows_b[b],
                         sem_g[b])

    def _gather_wait(b):
        pltpu.make_async_copy(h_hbm.at[cid].at[src_b[b].at[0]], rows_b[b],
                              sem_g[b]).wait()

    def _out_wait(b):
        pltpu.make_async_copy(rows_b[b], acc_s.at[dsts_b[b].at[0]],
                              sem_s[b]).wait()
        pltpu.make_async_copy(ee_b[b], den_s.at[dsts_b[b].at[0]],
                              sem_d[b]).wait()

    _idx_copy(0, 0)
    _idx_wait(0, 0)
    _gather_issue(0)
    _idx_copy(1, 1)

    def _trip(go, carry):
        for b in range(3):
            g = go * 3 + b
            bn = (b + 1) % 3

            @pl.when(g + 1 < NCHUNK)
            def _():
                _idx_wait(g + 1, bn)

                @pl.when(g >= 2)
                def _():
                    _out_wait(bn)
                _gather_issue(bn)

            _gather_wait(b)

            def _sp(cc, carry2, _b=b):
                off = pl.multiple_of(cc * 16, 16)
                s16 = src_b[_b][0, pl.ds(off, 16)]
                d16 = dst_b[_b][0, pl.ds(off, 16)]
                dsts_b[_b][0, pl.ds(off, 16)] = d16
                av = plsc.load_gather(asrc_v, [s16])
                bv = plsc.load_gather(adst_v, [d16])
                e = av + bv
                e = jnp.maximum(e, 0.2 * e)
                ee_b[_b][pl.ds(off, 16)] = jnp.exp(e - cvec)
                return carry2
            lax.fori_loop(0, CH // 16, _sp, 0)

            @pl.when(g + 2 < NCHUNK)
            def _():
                _idx_copy(g + 2, (b + 2) % 3)

            def _scale(gj, carry2, _b=b):
                base = pl.multiple_of(gj * 16, 16)
                eev = ee_b[_b][pl.ds(base, 16)]
                for j in range(16):
                    bc = eev.at[jnp.zeros((16,), _i32) + j].get(mode=_PIB)
                    er = base + j
                    for f in range(8):
                        sl = pl.ds(f * 16, 16)
                        rows_b[_b][er, sl] = rows_b[_b][er, sl] * bc
                return carry2
            lax.fori_loop(0, CH // 16, _scale, 0)

            pltpu.async_copy(rows_b[b], acc_s.at[dsts_b[b].at[0]], sem_s[b],
                             add=True)
            pltpu.async_copy(ee_b[b], den_s.at[dsts_b[b].at[0]], sem_d[b],
                             add=True)
        return carry
    lax.fori_loop(0, NCHUNK // 3, _trip, 0)
    for b in range(3):
        _out_wait(b)

    plsc.subcore_barrier()
    obase = pl.multiple_of(sid * ZROWS, 8)
    pltpu.sync_copy(acc_s.at[pl.ds(obase, ZROWS)],
                    acc_out.at[cid, pl.ds(obase, ZROWS)])

    @pl.when(sid == 0)
    def _():
        pltpu.sync_copy(den_s,
                        den_out.at[pl.ds(pl.multiple_of(cid * NPAD, 128),
                                         NPAD)])


_sc_conv = functools.partial(
    pl.kernel,
    out_type=(jax.ShapeDtypeStruct((2, NPAD, D), _f32),
              jax.ShapeDtypeStruct((2 * NPAD,), _f32)),
    mesh=plsc.VectorSubcoreMesh(core_axis_name="c", subcore_axis_name="s",
                                num_cores=2, num_subcores=16),
    compiler_params=pltpu.CompilerParams(needs_layout_passes=False),
    scratch_types=(
        pltpu.VMEM((NPAD,), _f32),
        pltpu.VMEM((NPAD,), _f32),
        pltpu.VMEM((640,), _f32),
        (pltpu.VMEM((1, CH), _i32),) * 3,
        (pltpu.VMEM((1, CH), _i32),) * 3,
        (pltpu.VMEM((1, CH), _i32),) * 3,
        (pltpu.VMEM((CH,), _f32),) * 3,
        (pltpu.VMEM((CH, D), _f32),) * 3,
        pltpu.VMEM((16,), _f32),
        pltpu.VMEM_SHARED((NPAD, D), _f32),
        pltpu.VMEM_SHARED((NPAD,), _f32),
        (pltpu.SemaphoreType.DMA,) * 3,
        (pltpu.SemaphoreType.DMA,) * 3,
        (pltpu.SemaphoreType.DMA,) * 3,
        (pltpu.SemaphoreType.DMA,) * 3,
        (pltpu.SemaphoreType.DMA,) * 3,
    ),
)(_sc_conv_body)


def _erf(z):
    a = jnp.abs(z)
    t = 1.0 / (1.0 + 0.3275911 * a)
    poly = t * (0.254829592 + t * (-0.284496736 + t * (1.421413741
                + t * (-1.453152027 + t * 1.061405429))))
    return jnp.sign(z) * (1.0 - poly * jnp.exp(-a * a))


def _gelu(v):
    return 0.5 * v * (1.0 + _erf(v * 0.7071067811865476))


def _ln(v, g, b):
    mu = jnp.mean(v, axis=-1, keepdims=True)
    c = v - mu
    var = jnp.mean(c * c, axis=-1, keepdims=True)
    return c * lax.rsqrt(var + 1e-12) * g + b


def _dot(a, b):
    return jnp.dot(a, b, preferred_element_type=_f32)


def _conv_prep(x2, gw_ref, gas_ref, gad_ref, as_ref, ad_ref,
               mas_ref, mad_ref, first):
    h = _dot(x2, gw_ref[...])
    a_s = jnp.sum(h * gas_ref[...], axis=-1, keepdims=True)
    a_d = jnp.sum(h * gad_ref[...], axis=-1, keepdims=True)
    as_ref[...] = a_s
    ad_ref[...] = a_d

    @pl.when(first)
    def _():
        mas_ref[...] = jnp.full((1, 1), -1e30, _f32)
        mad_ref[...] = jnp.full((1, 1), -1e30, _f32)
    mas_ref[...] = jnp.maximum(mas_ref[...], jnp.max(a_s))
    mad_ref[...] = jnp.maximum(mad_ref[...], jnp.max(a_d))
    return h


def _stage_a_body(x_ref, wpre_ref, bpre_ref,
                  gw0_ref, gas0_ref, gad0_ref, gw1_ref, gas1_ref, gad1_ref,
                  x1_ref, h_ref, as0_ref, ad0_ref, as1_ref, ad1_ref,
                  mas0_ref, mad0_ref, mas1_ref, mad1_ref):
    x = x_ref[...]
    t = _dot(x, wpre_ref[...]) + bpre_ref[...]
    x1 = jnp.maximum(t, 0.01 * t)
    x1_ref[...] = x1
    first = pl.program_id(0) == 0
    h0 = _conv_prep(x1, gw0_ref, gas0_ref, gad0_ref, as0_ref, ad0_ref,
                    mas0_ref, mad0_ref, first)
    h1 = _conv_prep(x1, gw1_ref, gas1_ref, gad1_ref, as1_ref, ad1_ref,
                    mas1_ref, mad1_ref, first)
    h_ref[...] = jnp.stack([h0, h1], axis=0)


def _stage_b_body(final, prex_ref, acc0_ref, den0_ref, acc1_ref, den1_ref,
                  gb0_ref, gb1_ref, wq_ref, bq_ref, wk_ref, bk_ref,
                  wv_ref, bv_ref, lng_ref, lnb_ref,
                  wm_ref, bm_ref, wo_ref, bo_ref, olg_ref, olb_ref,
                  *tail):
    prex = prex_ref[...]

    def _xt(acc_ref, den_ref, gb_ref):
        return acc_ref[...][0] / den_ref[...] + gb_ref[...]

    xt0 = _xt(acc0_ref, den0_ref, gb0_ref)
    xt1 = _xt(acc1_ref, den1_ref, gb1_ref)

    q = _dot(prex, wq_ref[...]) + bq_ref[...]
    k0 = _dot(xt0, wk_ref[...]) + bk_ref[...]
    k1 = _dot(xt1, wk_ref[...]) + bk_ref[...]
    l0 = jnp.sum(q * k0, axis=-1, keepdims=True)
    l1 = jnp.sum(q * k1, axis=-1, keepdims=True)
    m = jnp.maximum(l0, l1)
    w0 = jnp.exp(l0 - m)
    w1 = jnp.exp(l1 - m)
    v0 = _dot(xt0, wv_ref[...]) + bv_ref[...]
    v1 = _dot(xt1, wv_ref[...]) + bv_ref[...]
    xatt = (w0 * v0 + w1 * v1) / (w0 + w1)

    x = _ln(prex + _gelu(xatt), lng_ref[...], lnb_ref[...])
    midv = _gelu(_dot(x, wm_ref[...]) + bm_ref[...])
    mid2 = _dot(midv, wo_ref[...]) + bo_ref[...]
    x2 = _ln(x + mid2, olg_ref[...], olb_ref[...])

    if final:
        wfin_ref, bfin_ref, out_ref = tail
        out_ref[...] = _gelu(_dot(x2, wfin_ref[...]) + bfin_ref[...])
    else:
        (gw0_ref, gas0_ref, gad0_ref, gw1_ref, gas1_ref, gad1_ref,
         x2_ref, h_ref, as0_ref, ad0_ref, as1_ref, ad1_ref,
         mas0_ref, mad0_ref, mas1_ref, mad1_ref) = tail
        x2_ref[...] = x2
        first = pl.program_id(0) == 0
        h0 = _conv_prep(x2, gw0_ref, gas0_ref, gad0_ref, as0_ref, ad0_ref,
                        mas0_ref, mad0_ref, first)
        h1 = _conv_prep(x2, gw1_ref, gas1_ref, gad1_ref, as1_ref, ad1_ref,
                        mas1_ref, mad1_ref, first)
        h_ref[...] = jnp.stack([h0, h1], axis=0)


def _blk(shape, idx):
    return pl.BlockSpec(shape, idx)


_ROWB = _blk((R, D), lambda i: (i, 0))
_W128 = _blk((D, D), lambda i: (0, 0))
_ROW1 = _blk((1, D), lambda i: (0, 0))
_COL1 = _blk((R, 1), lambda i: (i, 0))
_SCLR = _blk((1, 1), lambda i: (0, 0))

_HSTK = _blk((2, R, D), lambda i: (0, i, 0))
_CONV_OUT_SHAPES = (
    jax.ShapeDtypeStruct((2, N, D), _f32),
    jax.ShapeDtypeStruct((N, 1), _f32),
    jax.ShapeDtypeStruct((N, 1), _f32),
    jax.ShapeDtypeStruct((N, 1), _f32),
    jax.ShapeDtypeStruct((N, 1), _f32),
)
_CONV_OUT_SPECS = (_HSTK, _COL1, _COL1, _COL1, _COL1)
_MAX_OUT = (jax.ShapeDtypeStruct((1, 1), _f32),) * 2
_MAX_SPEC = (_SCLR, _SCLR)


def _stage_a(x, wpre, bpre, gw0, gas0, gad0, gw1, gas1, gad1):
    return pl.pallas_call(
        _stage_a_body,
        grid=(GRID,),
        in_specs=[_ROWB, _W128, _ROW1,
                  _W128, _ROW1, _ROW1, _W128, _ROW1, _ROW1],
        out_specs=(_ROWB,) + _CONV_OUT_SPECS + _MAX_SPEC + _MAX_SPEC,
        out_shape=(jax.ShapeDtypeStruct((N, D), _f32),)
        + _CONV_OUT_SHAPES + _MAX_OUT + _MAX_OUT,
    )(x, wpre, bpre, gw0, gas0, gad0, gw1, gas1, gad1)


_ACC0 = _blk((1, R, D), lambda i: (0, i, 0))
_ACC1 = _blk((1, R, D), lambda i: (1, i, 0))
_DENB = _blk((R, 1), lambda i: (i, 0))
_WMID = _blk((D, MID), lambda i: (0, 0))
_ROWM = _blk((1, MID), lambda i: (0, 0))
_WOUT = _blk((MID, D), lambda i: (0, 0))

_B_COMMON_SPECS = [
    _ROWB, _ACC0, _DENB, _ACC1, _DENB,
    _ROW1, _ROW1, _W128, _ROW1, _W128, _ROW1, _W128, _ROW1,
    _ROW1, _ROW1, _WMID, _ROWM, _WOUT, _ROW1, _ROW1, _ROW1,
]


def _stage_b_mid(args, tail_weights):
    return pl.pallas_call(
        functools.partial(_stage_b_body, False),
        grid=(GRID,),
        in_specs=_B_COMMON_SPECS + [_W128, _ROW1, _ROW1, _W128, _ROW1, _ROW1],
        out_specs=(_ROWB,) + _CONV_OUT_SPECS + _MAX_SPEC + _MAX_SPEC,
        out_shape=(jax.ShapeDtypeStruct((N, D), _f32),)
        + _CONV_OUT_SHAPES + _MAX_OUT + _MAX_OUT,
    )(*args, *tail_weights)


def _stage_b_fin(args, tail_weights):
    return pl.pallas_call(
        functools.partial(_stage_b_body, True),
        grid=(GRID,),
        in_specs=_B_COMMON_SPECS + [_W128, _ROW1],
        out_specs=_ROWB,
        out_shape=jax.ShapeDtypeStruct((N, D), _f32),
    )(*args, *tail_weights)


def _edge_arrays(edge_index):
    ar = jnp.arange(N, dtype=_i32)
    pad_s = jnp.zeros((TOT - ETOT,), _i32)
    pad_d = jnp.full((TOT - ETOT,), N, _i32)
    srcs, dsts = [], []
    for j in range(2):
        src = jnp.concatenate([edge_index[j, 0].astype(_i32), ar, pad_s])
        dst = jnp.concatenate([edge_index[j, 1].astype(_i32), ar, pad_d])
        srcs.append(src.reshape(EROWS, CH))
        dsts.append(dst.reshape(EROWS, CH))
    return jnp.stack(srcs), jnp.stack(dsts)


def _table(a_s, a_d, mas, mad):
    c0 = mas[0, 0] + mad[0, 0]
    c = jnp.maximum(c0, 0.2 * c0)
    c16 = jnp.broadcast_to(c, (16,))
    asp = jnp.pad(a_s[:, 0], (0, NPAD - N))
    adp = jnp.pad(a_d[:, 0], (0, NPAD - N))
    return asp, adp, c16


def _run_convs(edges, cv0, cv1, h):
    as0, ad0, c0 = _table(*cv0)
    as1, ad1, c1 = _table(*cv1)
    acc, den = _sc_conv(edges[0], edges[1],
                        jnp.stack([as0, as1]), jnp.stack([ad0, ad1]),
                        jnp.stack([c0, c1]), h)
    den2 = den.reshape(2, NPAD)
    return (acc, den2[0, :N].reshape(N, 1), den2[1, :N].reshape(N, 1))


def kernel(x, params, edge_index):
    p = params

    def rowv(v):
        return v.reshape(1, -1).astype(_f32)

    edges = _edge_arrays(edge_index)

    def conv_w(i, j):
        return (p['gW_%d_%d' % (i, j)], rowv(p['gas_%d_%d' % (i, j)]),
                rowv(p['gad_%d_%d' % (i, j)]))

    (x1, h0s, as00, ad00, as01, ad01,
     mas00, mad00, mas01, mad01) = _stage_a(
        x, p['Wpre'], rowv(p['bpre']), *conv_w(0, 0), *conv_w(0, 1))

    acc0, den00, den01 = _run_convs(
        edges, (as00, ad00, mas00, mad00), (as01, ad01, mas01, mad01), h0s)

    def layer_args(i, prex, acc, den0, den1):
        temp = p['atemp_%d' % i]
        return (prex, acc, den0, acc, den1,
                rowv(p['gb_%d_0' % i]), rowv(p['gb_%d_1' % i]),
                p['aWq_%d' % i] * temp, rowv(p['abq_%d' % i]) * temp,
                p['aWk_%d' % i], rowv(p['abk_%d' % i]),
                p['aWv_%d' % i], rowv(p['abv_%d' % i]),
                rowv(p['ln_g']), rowv(p['ln_b']),
                p['oWm_%d' % i], rowv(p['obm_%d' % i]),
                p['oWo_%d' % i], rowv(p['obo_%d' % i]),
                rowv(p['olg_%d' % i]), rowv(p['olb_%d' % i]))

    (x2, h1s, as10, ad10, as11, ad11,
     mas10, mad10, mas11, mad11) = _stage_b_mid(
        layer_args(0, x1, acc0, den00, den01),
        conv_w(1, 0) + conv_w(1, 1))

    acc1, den10, den11 = _run_convs(
        edges, (as10, ad10, mas10, mad10), (as11, ad11, mas11, mad11), h1s)

    return _stage_b_fin(
        layer_args(1, x2, acc1, den10, den11),
        (p['Wfin'], rowv(p['bfin'])))
